# Initial kernel scaffold; baseline (speedup 1.0000x reference)
import jax
import jax.numpy as jnp
from jax.experimental import pallas as pl


def kernel(features, pos_edge_index, neg_edge_index, W_pos, attn_l_pos, attn_r_pos, b_pos, W_neg, attn_l_neg, attn_r_neg, b_neg, W1, b1, W2, b2):
    raise NotImplementedError("write your pallas kernel here")


# speedup vs baseline: 31.7419x; 31.7419x over previous
import functools

import jax
import jax.numpy as jnp
from jax import lax
from jax.experimental import pallas as pl
from jax.experimental.pallas import tpu as pltpu
from jax.experimental.pallas import tpu_sc as plsc

N = 10000
IN_DIM = 128
HID = 256
OUT_DIM = 128
H = 8
DH = HID // H
E = 160000

NC = 2
NS = 16
L = 16

NP = 10240
NPT = NP // NS
EPT = E // NS
C1 = 1000
C2 = 200
HALF = HID // 2


def _sc_mesh():
    return plsc.VectorSubcoreMesh(core_axis_name="c", subcore_axis_name="s",
                                  num_cores=NC, num_subcores=NS)


def _encode_body(x_ref, wp_ref, wn_ref, alp_ref, arp_ref, aln_ref, arn_ref,
                 fp0_ref, fp1_ref, fn0_ref, fn1_ref,
                 tlp_ref, trp_ref, tln_ref, trn_ref):
    x = x_ref[...]
    fp = jnp.dot(x, wp_ref[...], preferred_element_type=jnp.float32)
    fn = jnp.dot(x, wn_ref[...], preferred_element_type=jnp.float32)
    fp0_ref[...] = fp[:, :HALF]
    fp1_ref[...] = fp[:, HALF:]
    fn0_ref[...] = fn[:, :HALF]
    fn1_ref[...] = fn[:, HALF:]
    tlp_ref[...] = jnp.dot(fp, alp_ref[...], preferred_element_type=jnp.float32)
    trp_ref[...] = jnp.dot(fp, arp_ref[...], preferred_element_type=jnp.float32)
    tln_ref[...] = jnp.dot(fn, aln_ref[...], preferred_element_type=jnp.float32)
    trn_ref[...] = jnp.dot(fn, arn_ref[...], preferred_element_type=jnp.float32)


def _encode(x, wp, wn, alp, arp, aln, arn):
    R = 1000
    grid = (N // R,)
    bs_x = pl.BlockSpec((R, IN_DIM), lambda i: (i, 0))
    bs_w = pl.BlockSpec((IN_DIM, HID), lambda i: (0, 0))
    bs_a = pl.BlockSpec((HID, L), lambda i: (0, 0))
    bs_f = pl.BlockSpec((R, HALF), lambda i: (i, 0))
    bs_t = pl.BlockSpec((R, L), lambda i: (i, 0))
    outs = [jax.ShapeDtypeStruct((N, HALF), jnp.float32)] * 4 + \
           [jax.ShapeDtypeStruct((N, L), jnp.float32)] * 4
    return pl.pallas_call(
        _encode_body,
        grid=grid,
        in_specs=[bs_x, bs_w, bs_w, bs_a, bs_a, bs_a, bs_a],
        out_specs=[bs_f] * 4 + [bs_t] * 4,
        out_shape=outs,
    )(x, wp, wn, alp, arp, aln, arn)


def _pass1_body(src_p, dst_p, src_n, dst_n, tl_p, tr_p, tl_n, tr_n, z16,
                ex_p, ex_n, inv_p, inv_n,
                idx_s, idx_d, rl, rr, s_sh, sem1, sem2):
    c = lax.axis_index("c")
    s = lax.axis_index("s")

    pltpu.sync_copy(z16, s_sh.at[pl.ds(s * NPT, NPT)])
    plsc.subcore_barrier()

    def run(src_r, dst_r, tl_r, tr_r, ex_r):
        def chunk(k, _):
            base = s * EPT + k * C1
            pltpu.sync_copy(src_r.at[pl.ds(base, C1)], idx_s)
            pltpu.sync_copy(dst_r.at[pl.ds(base, C1)], idx_d)
            ca = pltpu.async_copy(tl_r.at[idx_s], rl, sem1)
            cb = pltpu.async_copy(tr_r.at[idx_d], rr, sem2)
            ca.wait()
            cb.wait()

            def row(i, _):
                v = rl[i, :] + rr[i, :]
                rl[i, :] = jnp.exp(jnp.maximum(v, 0.2 * v))
                return 0
            lax.fori_loop(0, C1, row, 0)
            pltpu.sync_copy(rl, ex_r.at[pl.ds(base, C1)])
            pltpu.sync_copy(rl, s_sh.at[idx_d], add=True)
            return 0
        lax.fori_loop(0, EPT // C1, chunk, 0)

    @pl.when(c == 0)
    def _():
        run(src_p, dst_p, tl_p, tr_p, ex_p)

    @pl.when(c == 1)
    def _():
        run(src_n, dst_n, tl_n, tr_n, ex_n)

    plsc.subcore_barrier()

    pltpu.sync_copy(s_sh.at[pl.ds(s * NPT, NPT)], rl.at[pl.ds(0, NPT)])

    def fin(i, _):
        rl[i, :] = 1.0 / jnp.maximum(rl[i, :], 1e-16)
        return 0
    lax.fori_loop(0, NPT, fin, 0)

    @pl.when(c == 0)
    def _():
        pltpu.sync_copy(rl.at[pl.ds(0, NPT)], inv_p.at[pl.ds(s * NPT, NPT)])

    @pl.when(c == 1)
    def _():
        pltpu.sync_copy(rl.at[pl.ds(0, NPT)], inv_n.at[pl.ds(s * NPT, NPT)])


def _pass1(src_p, dst_p, src_n, dst_n, tl_p, tr_p, tl_n, tr_n, z16):
    f = pl.kernel(
        _pass1_body,
        out_type=[jax.ShapeDtypeStruct((E, L), jnp.float32),
                  jax.ShapeDtypeStruct((E, L), jnp.float32),
                  jax.ShapeDtypeStruct((NP, L), jnp.float32),
                  jax.ShapeDtypeStruct((NP, L), jnp.float32)],
        mesh=_sc_mesh(),
        compiler_params=pltpu.CompilerParams(use_tc_tiling_on_sc=False),
        scratch_types=[
            pltpu.VMEM((C1,), jnp.int32),
            pltpu.VMEM((C1,), jnp.int32),
            pltpu.VMEM((C1, L), jnp.float32),
            pltpu.VMEM((C1, L), jnp.float32),
            pltpu.VMEM_SHARED((NP, L), jnp.float32),
            pltpu.SemaphoreType.DMA,
            pltpu.SemaphoreType.DMA,
        ],
    )
    return f(src_p, dst_p, src_n, dst_n, tl_p, tr_p, tl_n, tr_n, z16)


def _pass2_body(src, dst, ex, inv, f0, f1, z128,
                o0, o1,
                idx_s, idx_d, exb, invb, fb, osh, sem1, sem2):
    c = lax.axis_index("c")
    s = lax.axis_index("s")

    pltpu.sync_copy(z128, osh.at[pl.ds(s * NPT, NPT)])
    plsc.subcore_barrier()

    def run(f_r, o_r, head_base):
        col = [jnp.full((L,), head_base + hh, jnp.int32) for hh in range(4)]

        def chunk(k, _):
            base = s * EPT + k * C2
            pltpu.sync_copy(src.at[pl.ds(base, C2)], idx_s)
            pltpu.sync_copy(dst.at[pl.ds(base, C2)], idx_d)
            ca = pltpu.async_copy(inv.at[idx_d], invb, sem1)
            cb = pltpu.async_copy(f_r.at[idx_s], fb, sem2)
            pltpu.sync_copy(ex.at[pl.ds(base, C2)], exb)
            ca.wait()
            cb.wait()

            def row(i, _):
                al = exb[i, :] * invb[i, :]
                exb[i, :] = al
                for hh in range(4):
                    bc = plsc.load_gather(exb, [jnp.full((L,), 0, jnp.int32) + i,
                                                col[hh]])
                    for jj in range(2):
                        j = hh * 2 + jj
                        fv = fb[i, pl.ds(j * L, L)]
                        fb[i, pl.ds(j * L, L)] = fv * bc
                return 0
            lax.fori_loop(0, C2, row, 0)
            pltpu.sync_copy(fb, osh.at[idx_d], add=True)
            return 0
        lax.fori_loop(0, EPT // C2, chunk, 0)
        plsc.subcore_barrier()
        pltpu.sync_copy(osh.at[pl.ds(s * NPT, NPT)], o_r.at[pl.ds(s * NPT, NPT)])

    @pl.when(c == 0)
    def _():
        run(f0, o0, 0)

    @pl.when(c == 1)
    def _():
        run(f1, o1, 4)


def _pass2(src, dst, ex, inv, f0, f1, z128):
    f = pl.kernel(
        _pass2_body,
        out_type=[jax.ShapeDtypeStruct((NP, HALF), jnp.float32),
                  jax.ShapeDtypeStruct((NP, HALF), jnp.float32)],
        mesh=_sc_mesh(),
        compiler_params=pltpu.CompilerParams(use_tc_tiling_on_sc=False,
                                             needs_layout_passes=False),
        scratch_types=[
            pltpu.VMEM((C2,), jnp.int32),
            pltpu.VMEM((C2,), jnp.int32),
            pltpu.VMEM((C2, L), jnp.float32),
            pltpu.VMEM((C2, L), jnp.float32),
            pltpu.VMEM((C2, HALF), jnp.float32),
            pltpu.VMEM_SHARED((NP, HALF), jnp.float32),
            pltpu.SemaphoreType.DMA,
            pltpu.SemaphoreType.DMA,
        ],
    )
    return f(src, dst, ex, inv, f0, f1, z128)


def _mlp_body(op0_ref, op1_ref, on0_ref, on1_ref, bp_ref, bn_ref,
              w1_ref, b1_ref, w2_ref, b2_ref,
              hp_ref, hn_ref, hf_ref):
    hp0 = op0_ref[...] + bp_ref[0:1, :HALF]
    hp1 = op1_ref[...] + bp_ref[0:1, HALF:]
    hn0 = on0_ref[...] + bn_ref[0:1, :HALF]
    hn1 = on1_ref[...] + bn_ref[0:1, HALF:]
    hp_ref[:, :HALF] = hp0
    hp_ref[:, HALF:] = hp1
    hn_ref[:, :HALF] = hn0
    hn_ref[:, HALF:] = hn1
    w1 = w1_ref[...]
    z = (jnp.dot(hp0, w1[0:HALF, :], preferred_element_type=jnp.float32)
         + jnp.dot(hp1, w1[HALF:HID, :], preferred_element_type=jnp.float32)
         + jnp.dot(hn0, w1[HID:HID + HALF, :], preferred_element_type=jnp.float32)
         + jnp.dot(hn1, w1[HID + HALF:, :], preferred_element_type=jnp.float32)
         + b1_ref[0:1, :])
    z = jnp.maximum(z, 0.0)
    hf_ref[...] = jnp.dot(z, w2_ref[...], preferred_element_type=jnp.float32) + b2_ref[0:1, :]


def _mlp(op0, op1, on0, on1, bp, bn, w1, b1, w2, b2):
    R = 1000
    grid = (N // R,)
    bs_h = pl.BlockSpec((R, HALF), lambda i: (i, 0))
    return pl.pallas_call(
        _mlp_body,
        grid=grid,
        in_specs=[bs_h, bs_h, bs_h, bs_h,
                  pl.BlockSpec((1, HID), lambda i: (0, 0)),
                  pl.BlockSpec((1, HID), lambda i: (0, 0)),
                  pl.BlockSpec((2 * HID, HID), lambda i: (0, 0)),
                  pl.BlockSpec((1, HID), lambda i: (0, 0)),
                  pl.BlockSpec((HID, OUT_DIM), lambda i: (0, 0)),
                  pl.BlockSpec((1, OUT_DIM), lambda i: (0, 0))],
        out_specs=[pl.BlockSpec((R, HID), lambda i: (i, 0)),
                   pl.BlockSpec((R, HID), lambda i: (i, 0)),
                   pl.BlockSpec((R, OUT_DIM), lambda i: (i, 0))],
        out_shape=[jax.ShapeDtypeStruct((N, HID), jnp.float32),
                   jax.ShapeDtypeStruct((N, HID), jnp.float32),
                   jax.ShapeDtypeStruct((N, OUT_DIM), jnp.float32)],
    )(op0, op1, on0, on1, bp, bn, w1, b1, w2, b2)


def kernel(features, pos_edge_index, neg_edge_index, W_pos, attn_l_pos,
           attn_r_pos, b_pos, W_neg, attn_l_neg, attn_r_neg, b_neg,
           W1, b1, W2, b2):
    src_p = pos_edge_index[0].astype(jnp.int32)
    dst_p = pos_edge_index[1].astype(jnp.int32)
    src_n = neg_edge_index[0].astype(jnp.int32)
    dst_n = neg_edge_index[1].astype(jnp.int32)

    eye = jnp.eye(H, L, dtype=jnp.float32)
    alp = (attn_l_pos[:, :, None] * eye[:, None, :]).reshape(HID, L)
    arp = (attn_r_pos[:, :, None] * eye[:, None, :]).reshape(HID, L)
    aln = (attn_l_neg[:, :, None] * eye[:, None, :]).reshape(HID, L)
    arn = (attn_r_neg[:, :, None] * eye[:, None, :]).reshape(HID, L)

    z16 = jnp.zeros((NPT, L), jnp.float32)
    z128 = jnp.zeros((NPT, HALF), jnp.float32)

    fp0, fp1, fn0, fn1, tlp, trp, tln, trn = _encode(
        features, W_pos, W_neg, alp, arp, aln, arn)

    ex_p, ex_n, inv_p, inv_n = _pass1(
        src_p, dst_p, src_n, dst_n, tlp, trp, tln, trn, z16)

    op0, op1 = _pass2(src_p, dst_p, ex_p, inv_p, fp0, fp1, z128)
    on0, on1 = _pass2(src_n, dst_n, ex_n, inv_n, fn0, fn1, z128)
    op0, op1, on0, on1 = (a[:N] for a in (op0, op1, on0, on1))

    h_pos, h_neg, h_final = _mlp(
        op0, op1, on0, on1,
        b_pos.reshape(1, HID), b_neg.reshape(1, HID),
        W1, b1.reshape(1, HID), W2, b2.reshape(1, OUT_DIM))
    return (h_pos, h_neg, h_final)


# speedup vs baseline: 57.4473x; 1.8098x over previous
import functools

import jax
import jax.numpy as jnp
from jax import lax
from jax.experimental import pallas as pl
from jax.experimental.pallas import tpu as pltpu
from jax.experimental.pallas import tpu_sc as plsc

N = 10000
IN_DIM = 128
HID = 256
OUT_DIM = 128
H = 8
DH = HID // H
E = 160000

NC = 2
NS = 16
L = 16

NP = 10240
NPT = NP // NS
EPT = E // NS
C1 = 1000
C2 = 200
HALF = HID // 2


def _sc_mesh():
    return plsc.VectorSubcoreMesh(core_axis_name="c", subcore_axis_name="s",
                                  num_cores=NC, num_subcores=NS)


_BCAST_DNUMS = lax.GatherDimensionNumbers(
    offset_dims=(), collapsed_slice_dims=(0,), start_index_map=(0,))


def _bcast(v, idx):
    return lax.gather(v, idx[:, None], _BCAST_DNUMS, (1,),
                      mode=lax.GatherScatterMode.PROMISE_IN_BOUNDS)


def _encode_body(x_ref, wp_ref, wn_ref, alp_ref, arp_ref, aln_ref, arn_ref,
                 fp0_ref, fp1_ref, fn0_ref, fn1_ref,
                 tlp_ref, trp_ref, tln_ref, trn_ref):
    x = x_ref[...]
    fp = jnp.dot(x, wp_ref[...], preferred_element_type=jnp.float32)
    fn = jnp.dot(x, wn_ref[...], preferred_element_type=jnp.float32)
    fp0_ref[...] = fp[:, :HALF]
    fp1_ref[...] = fp[:, HALF:]
    fn0_ref[...] = fn[:, :HALF]
    fn1_ref[...] = fn[:, HALF:]
    tlp_ref[...] = jnp.dot(fp, alp_ref[...], preferred_element_type=jnp.float32)
    trp_ref[...] = jnp.dot(fp, arp_ref[...], preferred_element_type=jnp.float32)
    tln_ref[...] = jnp.dot(fn, aln_ref[...], preferred_element_type=jnp.float32)
    trn_ref[...] = jnp.dot(fn, arn_ref[...], preferred_element_type=jnp.float32)


def _encode(x, wp, wn, alp, arp, aln, arn):
    R = 1000
    grid = (N // R,)
    bs_x = pl.BlockSpec((R, IN_DIM), lambda i: (i, 0))
    bs_w = pl.BlockSpec((IN_DIM, HID), lambda i: (0, 0))
    bs_a = pl.BlockSpec((HID, L), lambda i: (0, 0))
    bs_f = pl.BlockSpec((R, HALF), lambda i: (i, 0))
    bs_t = pl.BlockSpec((R, L), lambda i: (i, 0))
    outs = [jax.ShapeDtypeStruct((N, HALF), jnp.float32)] * 4 + \
           [jax.ShapeDtypeStruct((N, L), jnp.float32)] * 4
    return pl.pallas_call(
        _encode_body,
        grid=grid,
        in_specs=[bs_x, bs_w, bs_w, bs_a, bs_a, bs_a, bs_a],
        out_specs=[bs_f] * 4 + [bs_t] * 4,
        out_shape=outs,
    )(x, wp, wn, alp, arp, aln, arn)


def _pass1_body(src_p, dst_p, src_n, dst_n, tl_p, tr_p, tl_n, tr_n, z16,
                ex_p, ex_n, inv_p, inv_n,
                idx_s, idx_d, rl, rr, s_sh, sem1, sem2):
    c = lax.axis_index("c")
    s = lax.axis_index("s")

    pltpu.sync_copy(z16, s_sh.at[pl.ds(s * NPT, NPT)])
    plsc.subcore_barrier()

    def run(src_r, dst_r, tl_r, tr_r, ex_r):
        def chunk(k, _):
            base = s * EPT + k * C1
            pltpu.sync_copy(src_r.at[pl.ds(base, C1)], idx_s)
            pltpu.sync_copy(dst_r.at[pl.ds(base, C1)], idx_d)
            ca = pltpu.async_copy(tl_r.at[idx_s], rl, sem1)
            cb = pltpu.async_copy(tr_r.at[idx_d], rr, sem2)
            ca.wait()
            cb.wait()

            @plsc.parallel_loop(0, C1, unroll=8)
            def row(i):
                v = rl[i, :] + rr[i, :]
                rl[i, :] = jnp.exp(jnp.maximum(v, 0.2 * v))
            pltpu.sync_copy(rl, ex_r.at[pl.ds(base, C1)])
            pltpu.sync_copy(rl, s_sh.at[idx_d], add=True)
            return 0
        lax.fori_loop(0, EPT // C1, chunk, 0)

    @pl.when(c == 0)
    def _():
        run(src_p, dst_p, tl_p, tr_p, ex_p)

    @pl.when(c == 1)
    def _():
        run(src_n, dst_n, tl_n, tr_n, ex_n)

    plsc.subcore_barrier()

    pltpu.sync_copy(s_sh.at[pl.ds(s * NPT, NPT)], rl.at[pl.ds(0, NPT)])

    @plsc.parallel_loop(0, NPT, unroll=8)
    def fin(i):
        rl[i, :] = 1.0 / jnp.maximum(rl[i, :], 1e-16)

    @pl.when(c == 0)
    def _():
        pltpu.sync_copy(rl.at[pl.ds(0, NPT)], inv_p.at[pl.ds(s * NPT, NPT)])

    @pl.when(c == 1)
    def _():
        pltpu.sync_copy(rl.at[pl.ds(0, NPT)], inv_n.at[pl.ds(s * NPT, NPT)])


def _pass1(src_p, dst_p, src_n, dst_n, tl_p, tr_p, tl_n, tr_n, z16):
    f = pl.kernel(
        _pass1_body,
        out_type=[jax.ShapeDtypeStruct((E, L), jnp.float32),
                  jax.ShapeDtypeStruct((E, L), jnp.float32),
                  jax.ShapeDtypeStruct((NP, L), jnp.float32),
                  jax.ShapeDtypeStruct((NP, L), jnp.float32)],
        mesh=_sc_mesh(),
        compiler_params=pltpu.CompilerParams(use_tc_tiling_on_sc=False),
        scratch_types=[
            pltpu.VMEM((C1,), jnp.int32),
            pltpu.VMEM((C1,), jnp.int32),
            pltpu.VMEM((C1, L), jnp.float32),
            pltpu.VMEM((C1, L), jnp.float32),
            pltpu.VMEM_SHARED((NP, L), jnp.float32),
            pltpu.SemaphoreType.DMA,
            pltpu.SemaphoreType.DMA,
        ],
    )
    return f(src_p, dst_p, src_n, dst_n, tl_p, tr_p, tl_n, tr_n, z16)


def _pass2_body(src, dst, ex, inv, f0, f1, z128,
                o0, o1,
                idx_s, idx_d, exb, invb, fb, osh, sem1, sem2):
    c = lax.axis_index("c")
    s = lax.axis_index("s")

    pltpu.sync_copy(z128, osh.at[pl.ds(s * NPT, NPT)])
    plsc.subcore_barrier()

    def run(f_r, o_r, head_base):
        col = [jnp.full((L,), head_base + hh, jnp.int32) for hh in range(4)]

        def chunk(k, _):
            base = s * EPT + k * C2
            pltpu.sync_copy(src.at[pl.ds(base, C2)], idx_s)
            pltpu.sync_copy(dst.at[pl.ds(base, C2)], idx_d)
            ca = pltpu.async_copy(inv.at[idx_d], invb, sem1)
            cb = pltpu.async_copy(f_r.at[idx_s], fb, sem2)
            pltpu.sync_copy(ex.at[pl.ds(base, C2)], exb)
            ca.wait()
            cb.wait()

            @plsc.parallel_loop(0, C2, unroll=4)
            def row(i):
                al = exb[i, :] * invb[i, :]
                for hh in range(4):
                    bc = _bcast(al, col[hh])
                    for jj in range(2):
                        j = hh * 2 + jj
                        fv = fb[i, pl.ds(j * L, L)]
                        fb[i, pl.ds(j * L, L)] = fv * bc
            pltpu.sync_copy(fb, osh.at[idx_d], add=True)
            return 0
        lax.fori_loop(0, EPT // C2, chunk, 0)
        plsc.subcore_barrier()
        pltpu.sync_copy(osh.at[pl.ds(s * NPT, NPT)], o_r.at[pl.ds(s * NPT, NPT)])

    @pl.when(c == 0)
    def _():
        run(f0, o0, 0)

    @pl.when(c == 1)
    def _():
        run(f1, o1, 4)


def _pass2(src, dst, ex, inv, f0, f1, z128):
    f = pl.kernel(
        _pass2_body,
        out_type=[jax.ShapeDtypeStruct((NP, HALF), jnp.float32),
                  jax.ShapeDtypeStruct((NP, HALF), jnp.float32)],
        mesh=_sc_mesh(),
        compiler_params=pltpu.CompilerParams(use_tc_tiling_on_sc=False,
                                             needs_layout_passes=False),
        scratch_types=[
            pltpu.VMEM((C2,), jnp.int32),
            pltpu.VMEM((C2,), jnp.int32),
            pltpu.VMEM((C2, L), jnp.float32),
            pltpu.VMEM((C2, L), jnp.float32),
            pltpu.VMEM((C2, HALF), jnp.float32),
            pltpu.VMEM_SHARED((NP, HALF), jnp.float32),
            pltpu.SemaphoreType.DMA,
            pltpu.SemaphoreType.DMA,
        ],
    )
    return f(src, dst, ex, inv, f0, f1, z128)


def _mlp_body(op0_ref, op1_ref, on0_ref, on1_ref, bp_ref, bn_ref,
              w1_ref, b1_ref, w2_ref, b2_ref,
              hp_ref, hn_ref, hf_ref):
    hp0 = op0_ref[...] + bp_ref[0:1, :HALF]
    hp1 = op1_ref[...] + bp_ref[0:1, HALF:]
    hn0 = on0_ref[...] + bn_ref[0:1, :HALF]
    hn1 = on1_ref[...] + bn_ref[0:1, HALF:]
    hp_ref[:, :HALF] = hp0
    hp_ref[:, HALF:] = hp1
    hn_ref[:, :HALF] = hn0
    hn_ref[:, HALF:] = hn1
    w1 = w1_ref[...]
    z = (jnp.dot(hp0, w1[0:HALF, :], preferred_element_type=jnp.float32)
         + jnp.dot(hp1, w1[HALF:HID, :], preferred_element_type=jnp.float32)
         + jnp.dot(hn0, w1[HID:HID + HALF, :], preferred_element_type=jnp.float32)
         + jnp.dot(hn1, w1[HID + HALF:, :], preferred_element_type=jnp.float32)
         + b1_ref[0:1, :])
    z = jnp.maximum(z, 0.0)
    hf_ref[...] = jnp.dot(z, w2_ref[...], preferred_element_type=jnp.float32) + b2_ref[0:1, :]


def _mlp(op0, op1, on0, on1, bp, bn, w1, b1, w2, b2):
    R = 1000
    grid = (N // R,)
    bs_h = pl.BlockSpec((R, HALF), lambda i: (i, 0))
    return pl.pallas_call(
        _mlp_body,
        grid=grid,
        in_specs=[bs_h, bs_h, bs_h, bs_h,
                  pl.BlockSpec((1, HID), lambda i: (0, 0)),
                  pl.BlockSpec((1, HID), lambda i: (0, 0)),
                  pl.BlockSpec((2 * HID, HID), lambda i: (0, 0)),
                  pl.BlockSpec((1, HID), lambda i: (0, 0)),
                  pl.BlockSpec((HID, OUT_DIM), lambda i: (0, 0)),
                  pl.BlockSpec((1, OUT_DIM), lambda i: (0, 0))],
        out_specs=[pl.BlockSpec((R, HID), lambda i: (i, 0)),
                   pl.BlockSpec((R, HID), lambda i: (i, 0)),
                   pl.BlockSpec((R, OUT_DIM), lambda i: (i, 0))],
        out_shape=[jax.ShapeDtypeStruct((N, HID), jnp.float32),
                   jax.ShapeDtypeStruct((N, HID), jnp.float32),
                   jax.ShapeDtypeStruct((N, OUT_DIM), jnp.float32)],
    )(op0, op1, on0, on1, bp, bn, w1, b1, w2, b2)


def kernel(features, pos_edge_index, neg_edge_index, W_pos, attn_l_pos,
           attn_r_pos, b_pos, W_neg, attn_l_neg, attn_r_neg, b_neg,
           W1, b1, W2, b2):
    src_p = pos_edge_index[0].astype(jnp.int32)
    dst_p = pos_edge_index[1].astype(jnp.int32)
    src_n = neg_edge_index[0].astype(jnp.int32)
    dst_n = neg_edge_index[1].astype(jnp.int32)

    eye = jnp.eye(H, L, dtype=jnp.float32)
    alp = (attn_l_pos[:, :, None] * eye[:, None, :]).reshape(HID, L)
    arp = (attn_r_pos[:, :, None] * eye[:, None, :]).reshape(HID, L)
    aln = (attn_l_neg[:, :, None] * eye[:, None, :]).reshape(HID, L)
    arn = (attn_r_neg[:, :, None] * eye[:, None, :]).reshape(HID, L)

    z16 = jnp.zeros((NPT, L), jnp.float32)
    z128 = jnp.zeros((NPT, HALF), jnp.float32)

    fp0, fp1, fn0, fn1, tlp, trp, tln, trn = _encode(
        features, W_pos, W_neg, alp, arp, aln, arn)

    ex_p, ex_n, inv_p, inv_n = _pass1(
        src_p, dst_p, src_n, dst_n, tlp, trp, tln, trn, z16)

    op0, op1 = _pass2(src_p, dst_p, ex_p, inv_p, fp0, fp1, z128)
    on0, on1 = _pass2(src_n, dst_n, ex_n, inv_n, fn0, fn1, z128)
    op0, op1, on0, on1 = (a[:N] for a in (op0, op1, on0, on1))

    h_pos, h_neg, h_final = _mlp(
        op0, op1, on0, on1,
        b_pos.reshape(1, HID), b_neg.reshape(1, HID),
        W1, b1.reshape(1, HID), W2, b2.reshape(1, OUT_DIM))
    return (h_pos, h_neg, h_final)


# speedup vs baseline: 88.9995x; 1.5492x over previous
import functools

import jax
import jax.numpy as jnp
from jax import lax
from jax.experimental import pallas as pl
from jax.experimental.pallas import tpu as pltpu
from jax.experimental.pallas import tpu_sc as plsc

N = 10000
IN_DIM = 128
HID = 256
OUT_DIM = 128
H = 8
DH = HID // H
E = 160000

NC = 2
NS = 16
L = 16

NP = 10240
NPT = NP // NS
EPT = E // NS
C1 = 1000
C2 = 80
HALF = HID // 2


def _sc_mesh():
    return plsc.VectorSubcoreMesh(core_axis_name="c", subcore_axis_name="s",
                                  num_cores=NC, num_subcores=NS)


_BCAST_DNUMS = lax.GatherDimensionNumbers(
    offset_dims=(), collapsed_slice_dims=(0,), start_index_map=(0,))


def _bcast(v, idx):
    return lax.gather(v, idx[:, None], _BCAST_DNUMS, (1,),
                      mode=lax.GatherScatterMode.PROMISE_IN_BOUNDS)


def _encode_body(x_ref, wp_ref, wn_ref, alp_ref, arp_ref, aln_ref, arn_ref,
                 fp0_ref, fp1_ref, fn0_ref, fn1_ref,
                 tlp_ref, trp_ref, tln_ref, trn_ref):
    x = x_ref[...]
    fp = jnp.dot(x, wp_ref[...], preferred_element_type=jnp.float32)
    fn = jnp.dot(x, wn_ref[...], preferred_element_type=jnp.float32)
    fp0_ref[...] = fp[:, :HALF]
    fp1_ref[...] = fp[:, HALF:]
    fn0_ref[...] = fn[:, :HALF]
    fn1_ref[...] = fn[:, HALF:]
    tlp_ref[...] = jnp.dot(fp, alp_ref[...], preferred_element_type=jnp.float32)
    trp_ref[...] = jnp.dot(fp, arp_ref[...], preferred_element_type=jnp.float32)
    tln_ref[...] = jnp.dot(fn, aln_ref[...], preferred_element_type=jnp.float32)
    trn_ref[...] = jnp.dot(fn, arn_ref[...], preferred_element_type=jnp.float32)


def _encode(x, wp, wn, alp, arp, aln, arn):
    R = 1000
    grid = (N // R,)
    bs_x = pl.BlockSpec((R, IN_DIM), lambda i: (i, 0))
    bs_w = pl.BlockSpec((IN_DIM, HID), lambda i: (0, 0))
    bs_a = pl.BlockSpec((HID, L), lambda i: (0, 0))
    bs_f = pl.BlockSpec((R, HALF), lambda i: (i, 0))
    bs_t = pl.BlockSpec((R, L), lambda i: (i, 0))
    outs = [jax.ShapeDtypeStruct((N, HALF), jnp.float32)] * 4 + \
           [jax.ShapeDtypeStruct((N, L), jnp.float32)] * 4
    return pl.pallas_call(
        _encode_body,
        grid=grid,
        in_specs=[bs_x, bs_w, bs_w, bs_a, bs_a, bs_a, bs_a],
        out_specs=[bs_f] * 4 + [bs_t] * 4,
        out_shape=outs,
    )(x, wp, wn, alp, arp, aln, arn)


def _pass1_body(src_p, dst_p, src_n, dst_n, tl_p, tr_p, tl_n, tr_n, z16,
                ex_p, ex_n, inv_p, inv_n,
                idx_s, idx_d, rl, rr, s_sh, sem1, sem2):
    c = lax.axis_index("c")
    s = lax.axis_index("s")

    pltpu.sync_copy(z16, s_sh.at[pl.ds(s * NPT, NPT)])
    plsc.subcore_barrier()

    def run(src_r, dst_r, tl_r, tr_r, ex_r):
        def chunk(k, _):
            base = s * EPT + k * C1
            ha = pltpu.async_copy(src_r.at[pl.ds(base, C1)], idx_s, sem1)
            hb = pltpu.async_copy(dst_r.at[pl.ds(base, C1)], idx_d, sem2)
            ha.wait()
            hb.wait()
            ca = pltpu.async_copy(tl_r.at[idx_s], rl, sem1)
            cb = pltpu.async_copy(tr_r.at[idx_d], rr, sem2)
            ca.wait()
            cb.wait()

            @plsc.parallel_loop(0, C1, unroll=8)
            def row(i):
                v = rl[i, :] + rr[i, :]
                rl[i, :] = jnp.exp(jnp.maximum(v, 0.2 * v))
            pltpu.sync_copy(rl, ex_r.at[pl.ds(base, C1)])
            pltpu.sync_copy(rl, s_sh.at[idx_d], add=True)
            return 0
        lax.fori_loop(0, EPT // C1, chunk, 0)

    @pl.when(c == 0)
    def _():
        run(src_p, dst_p, tl_p, tr_p, ex_p)

    @pl.when(c == 1)
    def _():
        run(src_n, dst_n, tl_n, tr_n, ex_n)

    plsc.subcore_barrier()

    pltpu.sync_copy(s_sh.at[pl.ds(s * NPT, NPT)], rl.at[pl.ds(0, NPT)])

    @plsc.parallel_loop(0, NPT, unroll=8)
    def fin(i):
        rl[i, :] = 1.0 / jnp.maximum(rl[i, :], 1e-16)

    @pl.when(c == 0)
    def _():
        pltpu.sync_copy(rl.at[pl.ds(0, NPT)], inv_p.at[pl.ds(s * NPT, NPT)])

    @pl.when(c == 1)
    def _():
        pltpu.sync_copy(rl.at[pl.ds(0, NPT)], inv_n.at[pl.ds(s * NPT, NPT)])


def _pass1(src_p, dst_p, src_n, dst_n, tl_p, tr_p, tl_n, tr_n, z16):
    f = pl.kernel(
        _pass1_body,
        out_type=[jax.ShapeDtypeStruct((E, L), jnp.float32),
                  jax.ShapeDtypeStruct((E, L), jnp.float32),
                  jax.ShapeDtypeStruct((NP, L), jnp.float32),
                  jax.ShapeDtypeStruct((NP, L), jnp.float32)],
        mesh=_sc_mesh(),
        compiler_params=pltpu.CompilerParams(use_tc_tiling_on_sc=False),
        scratch_types=[
            pltpu.VMEM((C1,), jnp.int32),
            pltpu.VMEM((C1,), jnp.int32),
            pltpu.VMEM((C1, L), jnp.float32),
            pltpu.VMEM((C1, L), jnp.float32),
            pltpu.VMEM_SHARED((NP, L), jnp.float32),
            pltpu.SemaphoreType.DMA,
            pltpu.SemaphoreType.DMA,
        ],
    )
    return f(src_p, dst_p, src_n, dst_n, tl_p, tr_p, tl_n, tr_n, z16)


NB = 3
NQ = 6
NCH = EPT // C2
GRP = 12
MAIN = (NCH - 5) // GRP


def _pass2_body(src, dst, ex, inv, f0, f1, z128,
                o0, o1,
                idxq_s, idxq_d, exb, invb, fb, osh, isem, gsem, ssem):
    c = lax.axis_index("c")
    s = lax.axis_index("s")

    pltpu.sync_copy(z128, osh.at[pl.ds(s * NPT, NPT)])
    plsc.subcore_barrier()

    def run(f_r, o_r, head_base):
        col = [jnp.full((L,), head_base + hh, jnp.int32) for hh in range(4)]
        tbase = s * EPT

        def idx_load(kk, q):
            pltpu.async_copy(src.at[pl.ds(tbase + kk * C2, C2)],
                             idxq_s.at[q], isem.at[q])
            pltpu.async_copy(dst.at[pl.ds(tbase + kk * C2, C2)],
                             idxq_d.at[q], isem.at[q])

        def idx_wait(q):
            pltpu.make_async_copy(src.at[pl.ds(0, C2)], idxq_s.at[q],
                                  isem.at[q]).wait()
            pltpu.make_async_copy(dst.at[pl.ds(0, C2)], idxq_d.at[q],
                                  isem.at[q]).wait()

        def gathers(kk, b, q):
            pltpu.async_copy(inv.at[idxq_d.at[q]], invb.at[b], gsem.at[b])
            pltpu.async_copy(f_r.at[idxq_s.at[q]], fb.at[b], gsem.at[b])
            pltpu.async_copy(ex.at[pl.ds(tbase + kk * C2, C2)], exb.at[b],
                             gsem.at[b])

        def gathers_wait(b):
            pltpu.make_async_copy(inv.at[pl.ds(0, C2)], invb.at[b],
                                  gsem.at[b]).wait()
            pltpu.make_async_copy(f_r.at[pl.ds(0, C2)], fb.at[b],
                                  gsem.at[b]).wait()
            pltpu.make_async_copy(ex.at[pl.ds(0, C2)], exb.at[b],
                                  gsem.at[b]).wait()

        def scatter(b, q):
            pltpu.async_copy(fb.at[b], osh.at[idxq_d.at[q]], ssem.at[b],
                             add=True)

        def scatter_wait(b, q):
            pltpu.make_async_copy(fb.at[b], osh.at[idxq_d.at[q]],
                                  ssem.at[b]).wait()

        def compute(b):
            @plsc.parallel_loop(0, C2, unroll=4)
            def row(i):
                al = exb[b, i, :] * invb[b, i, :]
                for hh in range(4):
                    bc = _bcast(al, col[hh])
                    for jj in range(2):
                        j = hh * 2 + jj
                        fv = fb[b, i, pl.ds(j * L, L)]
                        fb[b, i, pl.ds(j * L, L)] = fv * bc

        for q in range(4):
            idx_load(q, q)
        idx_wait(0)
        gathers(0, 0, 0)
        idx_wait(1)
        gathers(1, 1, 1)

        def phase(k, j, p_is_dyn, p=None):
            b, q = j % NB, j % NQ
            gathers_wait(b)
            compute(b)
            scatter(b, q)
            jw = (j - 1) % GRP
            if j >= 1:
                scatter_wait(jw % NB, jw % NQ)
            elif p_is_dyn:
                @pl.when(p > 0)
                def _():
                    scatter_wait(jw % NB, jw % NQ)
            j2 = (j + 2) % GRP
            idx_wait(j2 % NQ)
            gathers(k + 2, j2 % NB, j2 % NQ)
            idx_load(k + 4, (j + 4) % NQ)

        def grp(p, _):
            k0 = p * GRP
            for j in range(GRP):
                phase(k0 + j, j, True, p)
            return 0
        lax.fori_loop(0, MAIN, grp, 0)

        for k in range(MAIN * GRP, NCH):
            j = k % GRP
            b, q = j % NB, j % NQ
            gathers_wait(b)
            compute(b)
            scatter(b, q)
            jw = (j - 1) % GRP
            scatter_wait(jw % NB, jw % NQ)
            if k + 2 < NCH:
                j2 = (j + 2) % GRP
                idx_wait(j2 % NQ)
                gathers(k + 2, j2 % NB, j2 % NQ)
            if k + 4 < NCH:
                idx_load(k + 4, (j + 4) % NQ)
        jl = (NCH - 1) % GRP
        scatter_wait(jl % NB, jl % NQ)

        plsc.subcore_barrier()
        pltpu.sync_copy(osh.at[pl.ds(s * NPT, NPT)], o_r.at[pl.ds(s * NPT, NPT)])

    @pl.when(c == 0)
    def _():
        run(f0, o0, 0)

    @pl.when(c == 1)
    def _():
        run(f1, o1, 4)


def _pass2(src, dst, ex, inv, f0, f1, z128):
    f = pl.kernel(
        _pass2_body,
        out_type=[jax.ShapeDtypeStruct((NP, HALF), jnp.float32),
                  jax.ShapeDtypeStruct((NP, HALF), jnp.float32)],
        mesh=_sc_mesh(),
        compiler_params=pltpu.CompilerParams(use_tc_tiling_on_sc=False,
                                             needs_layout_passes=False),
        scratch_types=[
            pltpu.VMEM((NQ, C2), jnp.int32),
            pltpu.VMEM((NQ, C2), jnp.int32),
            pltpu.VMEM((NB, C2, L), jnp.float32),
            pltpu.VMEM((NB, C2, L), jnp.float32),
            pltpu.VMEM((NB, C2, HALF), jnp.float32),
            pltpu.VMEM_SHARED((NP, HALF), jnp.float32),
            pltpu.SemaphoreType.DMA((NQ,)),
            pltpu.SemaphoreType.DMA((NB,)),
            pltpu.SemaphoreType.DMA((NB,)),
        ],
    )
    return f(src, dst, ex, inv, f0, f1, z128)


def _mlp_body(op0_ref, op1_ref, on0_ref, on1_ref, bp_ref, bn_ref,
              w1_ref, b1_ref, w2_ref, b2_ref,
              hp_ref, hn_ref, hf_ref):
    hp0 = op0_ref[...] + bp_ref[0:1, :HALF]
    hp1 = op1_ref[...] + bp_ref[0:1, HALF:]
    hn0 = on0_ref[...] + bn_ref[0:1, :HALF]
    hn1 = on1_ref[...] + bn_ref[0:1, HALF:]
    hp_ref[:, :HALF] = hp0
    hp_ref[:, HALF:] = hp1
    hn_ref[:, :HALF] = hn0
    hn_ref[:, HALF:] = hn1
    w1 = w1_ref[...]
    z = (jnp.dot(hp0, w1[0:HALF, :], preferred_element_type=jnp.float32)
         + jnp.dot(hp1, w1[HALF:HID, :], preferred_element_type=jnp.float32)
         + jnp.dot(hn0, w1[HID:HID + HALF, :], preferred_element_type=jnp.float32)
         + jnp.dot(hn1, w1[HID + HALF:, :], preferred_element_type=jnp.float32)
         + b1_ref[0:1, :])
    z = jnp.maximum(z, 0.0)
    hf_ref[...] = jnp.dot(z, w2_ref[...], preferred_element_type=jnp.float32) + b2_ref[0:1, :]


def _mlp(op0, op1, on0, on1, bp, bn, w1, b1, w2, b2):
    R = 1000
    grid = (N // R,)
    bs_h = pl.BlockSpec((R, HALF), lambda i: (i, 0))
    return pl.pallas_call(
        _mlp_body,
        grid=grid,
        in_specs=[bs_h, bs_h, bs_h, bs_h,
                  pl.BlockSpec((1, HID), lambda i: (0, 0)),
                  pl.BlockSpec((1, HID), lambda i: (0, 0)),
                  pl.BlockSpec((2 * HID, HID), lambda i: (0, 0)),
                  pl.BlockSpec((1, HID), lambda i: (0, 0)),
                  pl.BlockSpec((HID, OUT_DIM), lambda i: (0, 0)),
                  pl.BlockSpec((1, OUT_DIM), lambda i: (0, 0))],
        out_specs=[pl.BlockSpec((R, HID), lambda i: (i, 0)),
                   pl.BlockSpec((R, HID), lambda i: (i, 0)),
                   pl.BlockSpec((R, OUT_DIM), lambda i: (i, 0))],
        out_shape=[jax.ShapeDtypeStruct((N, HID), jnp.float32),
                   jax.ShapeDtypeStruct((N, HID), jnp.float32),
                   jax.ShapeDtypeStruct((N, OUT_DIM), jnp.float32)],
    )(op0, op1, on0, on1, bp, bn, w1, b1, w2, b2)


def kernel(features, pos_edge_index, neg_edge_index, W_pos, attn_l_pos,
           attn_r_pos, b_pos, W_neg, attn_l_neg, attn_r_neg, b_neg,
           W1, b1, W2, b2):
    src_p = pos_edge_index[0].astype(jnp.int32)
    dst_p = pos_edge_index[1].astype(jnp.int32)
    src_n = neg_edge_index[0].astype(jnp.int32)
    dst_n = neg_edge_index[1].astype(jnp.int32)

    eye = jnp.eye(H, L, dtype=jnp.float32)
    alp = (attn_l_pos[:, :, None] * eye[:, None, :]).reshape(HID, L)
    arp = (attn_r_pos[:, :, None] * eye[:, None, :]).reshape(HID, L)
    aln = (attn_l_neg[:, :, None] * eye[:, None, :]).reshape(HID, L)
    arn = (attn_r_neg[:, :, None] * eye[:, None, :]).reshape(HID, L)

    z16 = jnp.zeros((NPT, L), jnp.float32)
    z128 = jnp.zeros((NPT, HALF), jnp.float32)

    fp0, fp1, fn0, fn1, tlp, trp, tln, trn = _encode(
        features, W_pos, W_neg, alp, arp, aln, arn)

    ex_p, ex_n, inv_p, inv_n = _pass1(
        src_p, dst_p, src_n, dst_n, tlp, trp, tln, trn, z16)

    op0, op1 = _pass2(src_p, dst_p, ex_p, inv_p, fp0, fp1, z128)
    on0, on1 = _pass2(src_n, dst_n, ex_n, inv_n, fn0, fn1, z128)
    op0, op1, on0, on1 = (a[:N] for a in (op0, op1, on0, on1))

    h_pos, h_neg, h_final = _mlp(
        op0, op1, on0, on1,
        b_pos.reshape(1, HID), b_neg.reshape(1, HID),
        W1, b1.reshape(1, HID), W2, b2.reshape(1, OUT_DIM))
    return (h_pos, h_neg, h_final)


# speedup vs baseline: 93.4721x; 1.0503x over previous
import functools

import jax
import jax.numpy as jnp
from jax import lax
from jax.experimental import pallas as pl
from jax.experimental.pallas import tpu as pltpu
from jax.experimental.pallas import tpu_sc as plsc

N = 10000
IN_DIM = 128
HID = 256
OUT_DIM = 128
H = 8
DH = HID // H
E = 160000

NC = 2
NS = 16
L = 16

NP = 10240
NPT = NP // NS
EPT = E // NS
C1 = 1000
C2 = 80
HALF = HID // 2


def _sc_mesh():
    return plsc.VectorSubcoreMesh(core_axis_name="c", subcore_axis_name="s",
                                  num_cores=NC, num_subcores=NS)


_BCAST_DNUMS = lax.GatherDimensionNumbers(
    offset_dims=(), collapsed_slice_dims=(0,), start_index_map=(0,))


def _bcast(v, idx):
    return lax.gather(v, idx[:, None], _BCAST_DNUMS, (1,),
                      mode=lax.GatherScatterMode.PROMISE_IN_BOUNDS)


def _encode_body(x_ref, wp_ref, wn_ref, alp_ref, arp_ref, aln_ref, arn_ref,
                 fp0_ref, fp1_ref, fn0_ref, fn1_ref,
                 tlp_ref, trp_ref, tln_ref, trn_ref):
    x = x_ref[...]
    fp = jnp.dot(x, wp_ref[...], preferred_element_type=jnp.float32)
    fn = jnp.dot(x, wn_ref[...], preferred_element_type=jnp.float32)
    fp0_ref[...] = fp[:, :HALF]
    fp1_ref[...] = fp[:, HALF:]
    fn0_ref[...] = fn[:, :HALF]
    fn1_ref[...] = fn[:, HALF:]
    tlp_ref[...] = jnp.dot(fp, alp_ref[...], preferred_element_type=jnp.float32)
    trp_ref[...] = jnp.dot(fp, arp_ref[...], preferred_element_type=jnp.float32)
    tln_ref[...] = jnp.dot(fn, aln_ref[...], preferred_element_type=jnp.float32)
    trn_ref[...] = jnp.dot(fn, arn_ref[...], preferred_element_type=jnp.float32)


def _encode(x, wp, wn, alp, arp, aln, arn):
    R = 1000
    grid = (N // R,)
    bs_x = pl.BlockSpec((R, IN_DIM), lambda i: (i, 0))
    bs_w = pl.BlockSpec((IN_DIM, HID), lambda i: (0, 0))
    bs_a = pl.BlockSpec((HID, L), lambda i: (0, 0))
    bs_f = pl.BlockSpec((R, HALF), lambda i: (i, 0))
    bs_t = pl.BlockSpec((R, L), lambda i: (i, 0))
    outs = [jax.ShapeDtypeStruct((N, HALF), jnp.float32)] * 4 + \
           [jax.ShapeDtypeStruct((N, L), jnp.float32)] * 4
    return pl.pallas_call(
        _encode_body,
        grid=grid,
        in_specs=[bs_x, bs_w, bs_w, bs_a, bs_a, bs_a, bs_a],
        out_specs=[bs_f] * 4 + [bs_t] * 4,
        out_shape=outs,
    )(x, wp, wn, alp, arp, aln, arn)


def _pass1_body(src_p, dst_p, src_n, dst_n, tl_p, tr_p, tl_n, tr_n, z16,
                ex_p, ex_n, inv_p, inv_n,
                idx_s, idx_d, rl, rr, s_sh, sem1, sem2):
    c = lax.axis_index("c")
    s = lax.axis_index("s")

    pltpu.sync_copy(z16, s_sh.at[pl.ds(s * NPT, NPT)])
    plsc.subcore_barrier()

    def run(src_r, dst_r, tl_r, tr_r, ex_r):
        def chunk(k, _):
            base = s * EPT + k * C1
            ha = pltpu.async_copy(src_r.at[pl.ds(base, C1)], idx_s, sem1)
            hb = pltpu.async_copy(dst_r.at[pl.ds(base, C1)], idx_d, sem2)
            ha.wait()
            hb.wait()
            ca = pltpu.async_copy(tl_r.at[idx_s], rl, sem1)
            cb = pltpu.async_copy(tr_r.at[idx_d], rr, sem2)
            ca.wait()
            cb.wait()

            @plsc.parallel_loop(0, C1, unroll=8)
            def row(i):
                v = rl[i, :] + rr[i, :]
                rl[i, :] = jnp.exp(jnp.maximum(v, 0.2 * v))
            pltpu.sync_copy(rl, ex_r.at[pl.ds(base, C1)])
            pltpu.sync_copy(rl, s_sh.at[idx_d], add=True)
            return 0
        lax.fori_loop(0, EPT // C1, chunk, 0)

    @pl.when(c == 0)
    def _():
        run(src_p, dst_p, tl_p, tr_p, ex_p)

    @pl.when(c == 1)
    def _():
        run(src_n, dst_n, tl_n, tr_n, ex_n)

    plsc.subcore_barrier()

    pltpu.sync_copy(s_sh.at[pl.ds(s * NPT, NPT)], rl.at[pl.ds(0, NPT)])

    @plsc.parallel_loop(0, NPT, unroll=8)
    def fin(i):
        rl[i, :] = 1.0 / jnp.maximum(rl[i, :], 1e-16)

    @pl.when(c == 0)
    def _():
        pltpu.sync_copy(rl.at[pl.ds(0, NPT)], inv_p.at[pl.ds(s * NPT, NPT)])

    @pl.when(c == 1)
    def _():
        pltpu.sync_copy(rl.at[pl.ds(0, NPT)], inv_n.at[pl.ds(s * NPT, NPT)])


def _pass1(src_p, dst_p, src_n, dst_n, tl_p, tr_p, tl_n, tr_n, z16):
    f = pl.kernel(
        _pass1_body,
        out_type=[jax.ShapeDtypeStruct((E, L), jnp.float32),
                  jax.ShapeDtypeStruct((E, L), jnp.float32),
                  jax.ShapeDtypeStruct((NP, L), jnp.float32),
                  jax.ShapeDtypeStruct((NP, L), jnp.float32)],
        mesh=_sc_mesh(),
        compiler_params=pltpu.CompilerParams(use_tc_tiling_on_sc=False),
        scratch_types=[
            pltpu.VMEM((C1,), jnp.int32),
            pltpu.VMEM((C1,), jnp.int32),
            pltpu.VMEM((C1, L), jnp.float32),
            pltpu.VMEM((C1, L), jnp.float32),
            pltpu.VMEM_SHARED((NP, L), jnp.float32),
            pltpu.SemaphoreType.DMA,
            pltpu.SemaphoreType.DMA,
        ],
    )
    return f(src_p, dst_p, src_n, dst_n, tl_p, tr_p, tl_n, tr_n, z16)


NB = 3
NQ = 6
NCH = EPT // C2
GRP = 6
MAIN = (NCH - 5) // GRP


def _pass2_body(src_p, dst_p, src_n, dst_n, ex_p, ex_n, inv_p, inv_n,
                fp0, fp1, fn0, fn1, z128,
                op0, op1, on0, on1,
                idxq_s, idxq_d, exb, invb, fb, osh, isem, gsem, ssem):
    c = lax.axis_index("c")
    s = lax.axis_index("s")

    def run(src, dst, ex, inv, f_r, o_r, head_base):
        col = [jnp.full((L,), head_base + hh, jnp.int32) for hh in range(4)]
        tbase = s * EPT

        def idx_load(kk, q):
            pltpu.async_copy(src.at[pl.ds(tbase + kk * C2, C2)],
                             idxq_s.at[q], isem.at[q])
            pltpu.async_copy(dst.at[pl.ds(tbase + kk * C2, C2)],
                             idxq_d.at[q], isem.at[q])

        def idx_wait(q):
            pltpu.make_async_copy(src.at[pl.ds(0, C2)], idxq_s.at[q],
                                  isem.at[q]).wait()
            pltpu.make_async_copy(dst.at[pl.ds(0, C2)], idxq_d.at[q],
                                  isem.at[q]).wait()

        def gathers(kk, b, q):
            pltpu.async_copy(inv.at[idxq_d.at[q]], invb.at[b], gsem.at[b])
            pltpu.async_copy(f_r.at[idxq_s.at[q]], fb.at[b], gsem.at[b])
            pltpu.async_copy(ex.at[pl.ds(tbase + kk * C2, C2)], exb.at[b],
                             gsem.at[b])

        def gathers_wait(b):
            pltpu.make_async_copy(inv.at[pl.ds(0, C2)], invb.at[b],
                                  gsem.at[b]).wait()
            pltpu.make_async_copy(f_r.at[pl.ds(0, C2)], fb.at[b],
                                  gsem.at[b]).wait()
            pltpu.make_async_copy(ex.at[pl.ds(0, C2)], exb.at[b],
                                  gsem.at[b]).wait()

        def scatter(b, q):
            pltpu.async_copy(fb.at[b], osh.at[idxq_d.at[q]], ssem.at[b],
                             add=True)

        def scatter_wait(b, q):
            pltpu.make_async_copy(fb.at[b], osh.at[idxq_d.at[q]],
                                  ssem.at[b]).wait()

        def compute(b):
            @plsc.parallel_loop(0, C2, unroll=4)
            def row(i):
                al = exb[b, i, :] * invb[b, i, :]
                for hh in range(4):
                    bc = _bcast(al, col[hh])
                    for jj in range(2):
                        j = hh * 2 + jj
                        fv = fb[b, i, pl.ds(j * L, L)]
                        fb[b, i, pl.ds(j * L, L)] = fv * bc

        for q in range(4):
            idx_load(q, q)
        idx_wait(0)
        gathers(0, 0, 0)
        idx_wait(1)
        gathers(1, 1, 1)

        def phase(k, j, p_is_dyn, p=None):
            b, q = j % NB, j % NQ
            gathers_wait(b)
            compute(b)
            scatter(b, q)
            jw = (j - 1) % GRP
            if j >= 1:
                scatter_wait(jw % NB, jw % NQ)
            elif p_is_dyn:
                @pl.when(p > 0)
                def _():
                    scatter_wait(jw % NB, jw % NQ)
            j2 = (j + 2) % GRP
            idx_wait(j2 % NQ)
            gathers(k + 2, j2 % NB, j2 % NQ)
            idx_load(k + 4, (j + 4) % NQ)

        def grp(p, _):
            k0 = p * GRP
            for j in range(GRP):
                phase(k0 + j, j, True, p)
            return 0
        lax.fori_loop(0, MAIN, grp, 0)

        for k in range(MAIN * GRP, NCH):
            j = k % GRP
            b, q = j % NB, j % NQ
            gathers_wait(b)
            compute(b)
            scatter(b, q)
            jw = (j - 1) % GRP
            scatter_wait(jw % NB, jw % NQ)
            if k + 2 < NCH:
                j2 = (j + 2) % GRP
                idx_wait(j2 % NQ)
                gathers(k + 2, j2 % NB, j2 % NQ)
            if k + 4 < NCH:
                idx_load(k + 4, (j + 4) % NQ)
        jl = (NCH - 1) % GRP
        scatter_wait(jl % NB, jl % NQ)

        plsc.subcore_barrier()
        pltpu.sync_copy(osh.at[pl.ds(s * NPT, NPT)], o_r.at[pl.ds(s * NPT, NPT)])


    for (srcr, dstr, exr, invr, fh0, fh1, oh0, oh1) in (
            (src_p, dst_p, ex_p, inv_p, fp0, fp1, op0, op1),
            (src_n, dst_n, ex_n, inv_n, fn0, fn1, on0, on1)):
        pltpu.sync_copy(z128, osh.at[pl.ds(s * NPT, NPT)])
        plsc.subcore_barrier()

        @pl.when(c == 0)
        def _():
            run(srcr, dstr, exr, invr, fh0, oh0, 0)

        @pl.when(c == 1)
        def _():
            run(srcr, dstr, exr, invr, fh1, oh1, 4)

        plsc.subcore_barrier()


def _pass2(src_p, dst_p, src_n, dst_n, ex_p, ex_n, inv_p, inv_n,
           fp0, fp1, fn0, fn1, z128):
    f = pl.kernel(
        _pass2_body,
        out_type=[jax.ShapeDtypeStruct((NP, HALF), jnp.float32)] * 4,
        mesh=_sc_mesh(),
        compiler_params=pltpu.CompilerParams(use_tc_tiling_on_sc=False,
                                             needs_layout_passes=False),
        scratch_types=[
            pltpu.VMEM((NQ, C2), jnp.int32),
            pltpu.VMEM((NQ, C2), jnp.int32),
            pltpu.VMEM((NB, C2, L), jnp.float32),
            pltpu.VMEM((NB, C2, L), jnp.float32),
            pltpu.VMEM((NB, C2, HALF), jnp.float32),
            pltpu.VMEM_SHARED((NP, HALF), jnp.float32),
            pltpu.SemaphoreType.DMA((NQ,)),
            pltpu.SemaphoreType.DMA((NB,)),
            pltpu.SemaphoreType.DMA((NB,)),
        ],
    )
    return f(src_p, dst_p, src_n, dst_n, ex_p, ex_n, inv_p, inv_n,
             fp0, fp1, fn0, fn1, z128)


def _mlp_body(op0_ref, op1_ref, on0_ref, on1_ref, bp_ref, bn_ref,
              w1_ref, b1_ref, w2_ref, b2_ref,
              hp_ref, hn_ref, hf_ref):
    hp0 = op0_ref[...] + bp_ref[0:1, :HALF]
    hp1 = op1_ref[...] + bp_ref[0:1, HALF:]
    hn0 = on0_ref[...] + bn_ref[0:1, :HALF]
    hn1 = on1_ref[...] + bn_ref[0:1, HALF:]
    hp_ref[:, :HALF] = hp0
    hp_ref[:, HALF:] = hp1
    hn_ref[:, :HALF] = hn0
    hn_ref[:, HALF:] = hn1
    w1 = w1_ref[...]
    z = (jnp.dot(hp0, w1[0:HALF, :], preferred_element_type=jnp.float32)
         + jnp.dot(hp1, w1[HALF:HID, :], preferred_element_type=jnp.float32)
         + jnp.dot(hn0, w1[HID:HID + HALF, :], preferred_element_type=jnp.float32)
         + jnp.dot(hn1, w1[HID + HALF:, :], preferred_element_type=jnp.float32)
         + b1_ref[0:1, :])
    z = jnp.maximum(z, 0.0)
    hf_ref[...] = jnp.dot(z, w2_ref[...], preferred_element_type=jnp.float32) + b2_ref[0:1, :]


def _mlp(op0, op1, on0, on1, bp, bn, w1, b1, w2, b2):
    R = 1000
    grid = (N // R,)
    bs_h = pl.BlockSpec((R, HALF), lambda i: (i, 0))
    return pl.pallas_call(
        _mlp_body,
        grid=grid,
        in_specs=[bs_h, bs_h, bs_h, bs_h,
                  pl.BlockSpec((1, HID), lambda i: (0, 0)),
                  pl.BlockSpec((1, HID), lambda i: (0, 0)),
                  pl.BlockSpec((2 * HID, HID), lambda i: (0, 0)),
                  pl.BlockSpec((1, HID), lambda i: (0, 0)),
                  pl.BlockSpec((HID, OUT_DIM), lambda i: (0, 0)),
                  pl.BlockSpec((1, OUT_DIM), lambda i: (0, 0))],
        out_specs=[pl.BlockSpec((R, HID), lambda i: (i, 0)),
                   pl.BlockSpec((R, HID), lambda i: (i, 0)),
                   pl.BlockSpec((R, OUT_DIM), lambda i: (i, 0))],
        out_shape=[jax.ShapeDtypeStruct((N, HID), jnp.float32),
                   jax.ShapeDtypeStruct((N, HID), jnp.float32),
                   jax.ShapeDtypeStruct((N, OUT_DIM), jnp.float32)],
    )(op0, op1, on0, on1, bp, bn, w1, b1, w2, b2)


def kernel(features, pos_edge_index, neg_edge_index, W_pos, attn_l_pos,
           attn_r_pos, b_pos, W_neg, attn_l_neg, attn_r_neg, b_neg,
           W1, b1, W2, b2):
    src_p = pos_edge_index[0].astype(jnp.int32)
    dst_p = pos_edge_index[1].astype(jnp.int32)
    src_n = neg_edge_index[0].astype(jnp.int32)
    dst_n = neg_edge_index[1].astype(jnp.int32)

    eye = jnp.eye(H, L, dtype=jnp.float32)
    alp = (attn_l_pos[:, :, None] * eye[:, None, :]).reshape(HID, L)
    arp = (attn_r_pos[:, :, None] * eye[:, None, :]).reshape(HID, L)
    aln = (attn_l_neg[:, :, None] * eye[:, None, :]).reshape(HID, L)
    arn = (attn_r_neg[:, :, None] * eye[:, None, :]).reshape(HID, L)

    z16 = jnp.zeros((NPT, L), jnp.float32)
    z128 = jnp.zeros((NPT, HALF), jnp.float32)

    fp0, fp1, fn0, fn1, tlp, trp, tln, trn = _encode(
        features, W_pos, W_neg, alp, arp, aln, arn)

    ex_p, ex_n, inv_p, inv_n = _pass1(
        src_p, dst_p, src_n, dst_n, tlp, trp, tln, trn, z16)

    op0, op1, on0, on1 = _pass2(src_p, dst_p, src_n, dst_n, ex_p, ex_n,
                                inv_p, inv_n, fp0, fp1, fn0, fn1, z128)

    h_pos, h_neg, h_final = _mlp(
        op0, op1, on0, on1,
        b_pos.reshape(1, HID), b_neg.reshape(1, HID),
        W1, b1.reshape(1, HID), W2, b2.reshape(1, OUT_DIM))
    return (h_pos, h_neg, h_final)


# speedup vs baseline: 97.2536x; 1.0405x over previous
import functools

import jax
import jax.numpy as jnp
from jax import lax
from jax.experimental import pallas as pl
from jax.experimental.pallas import tpu as pltpu
from jax.experimental.pallas import tpu_sc as plsc

N = 10000
IN_DIM = 128
HID = 256
OUT_DIM = 128
H = 8
DH = HID // H
E = 160000

NC = 2
NS = 16
L = 16

NP = 10240
NPT = NP // NS
EPT = E // NS
C1 = 400
C2 = 80
HALF = HID // 2


def _sc_mesh():
    return plsc.VectorSubcoreMesh(core_axis_name="c", subcore_axis_name="s",
                                  num_cores=NC, num_subcores=NS)


_BCAST_DNUMS = lax.GatherDimensionNumbers(
    offset_dims=(), collapsed_slice_dims=(0,), start_index_map=(0,))


def _bcast(v, idx):
    return lax.gather(v, idx[:, None], _BCAST_DNUMS, (1,),
                      mode=lax.GatherScatterMode.PROMISE_IN_BOUNDS)


def _encode_body(x_ref, wp_ref, wn_ref, alp_ref, arp_ref, aln_ref, arn_ref,
                 fp0_ref, fp1_ref, fn0_ref, fn1_ref,
                 tlp_ref, trp_ref, tln_ref, trn_ref):
    x = x_ref[...]
    fp = jnp.dot(x, wp_ref[...], preferred_element_type=jnp.float32)
    fn = jnp.dot(x, wn_ref[...], preferred_element_type=jnp.float32)
    fp0_ref[...] = fp[:, :HALF]
    fp1_ref[...] = fp[:, HALF:]
    fn0_ref[...] = fn[:, :HALF]
    fn1_ref[...] = fn[:, HALF:]
    tlp_ref[...] = jnp.dot(fp, alp_ref[...], preferred_element_type=jnp.float32)
    trp_ref[...] = jnp.dot(fp, arp_ref[...], preferred_element_type=jnp.float32)
    tln_ref[...] = jnp.dot(fn, aln_ref[...], preferred_element_type=jnp.float32)
    trn_ref[...] = jnp.dot(fn, arn_ref[...], preferred_element_type=jnp.float32)


def _encode(x, wp, wn, alp, arp, aln, arn):
    R = 1000
    grid = (N // R,)
    bs_x = pl.BlockSpec((R, IN_DIM), lambda i: (i, 0))
    bs_w = pl.BlockSpec((IN_DIM, HID), lambda i: (0, 0))
    bs_a = pl.BlockSpec((HID, L), lambda i: (0, 0))
    bs_f = pl.BlockSpec((R, HALF), lambda i: (i, 0))
    bs_t = pl.BlockSpec((R, L), lambda i: (i, 0))
    outs = [jax.ShapeDtypeStruct((N, HALF), jnp.float32)] * 4 + \
           [jax.ShapeDtypeStruct((N, L), jnp.float32)] * 4
    return pl.pallas_call(
        _encode_body,
        grid=grid,
        in_specs=[bs_x, bs_w, bs_w, bs_a, bs_a, bs_a, bs_a],
        out_specs=[bs_f] * 4 + [bs_t] * 4,
        out_shape=outs,
    )(x, wp, wn, alp, arp, aln, arn)


C1N = EPT // C1
M1 = (C1N - 5) // 6


def _pass1_body(src_p, dst_p, src_n, dst_n, tl_p, tr_p, tl_n, tr_n, z16,
                ex_p, ex_n, inv_p, inv_n,
                idxq_s, idxq_d, rl, rr, fin_b, s_sh, isem, gsem, wsem, ssem):
    c = lax.axis_index("c")
    s = lax.axis_index("s")

    pltpu.sync_copy(z16, s_sh.at[pl.ds(s * NPT, NPT)])
    plsc.subcore_barrier()

    def run(src_r, dst_r, tl_r, tr_r, ex_r):
        tbase = s * EPT

        def idx_load(kk, q):
            pltpu.async_copy(src_r.at[pl.ds(tbase + kk * C1, C1)],
                             idxq_s.at[q], isem.at[q])
            pltpu.async_copy(dst_r.at[pl.ds(tbase + kk * C1, C1)],
                             idxq_d.at[q], isem.at[q])

        def idx_wait(q):
            pltpu.make_async_copy(src_r.at[pl.ds(0, C1)], idxq_s.at[q],
                                  isem.at[q]).wait()
            pltpu.make_async_copy(dst_r.at[pl.ds(0, C1)], idxq_d.at[q],
                                  isem.at[q]).wait()

        def gathers(b, q):
            pltpu.async_copy(tl_r.at[idxq_s.at[q]], rl.at[b], gsem.at[b])
            pltpu.async_copy(tr_r.at[idxq_d.at[q]], rr.at[b], gsem.at[b])

        def gathers_wait(b):
            pltpu.make_async_copy(tl_r.at[pl.ds(0, C1)], rl.at[b],
                                  gsem.at[b]).wait()
            pltpu.make_async_copy(tr_r.at[pl.ds(0, C1)], rr.at[b],
                                  gsem.at[b]).wait()

        def stores(kk, b, q):
            pltpu.async_copy(rl.at[b], ex_r.at[pl.ds(tbase + kk * C1, C1)],
                             wsem.at[b])
            pltpu.async_copy(rl.at[b], s_sh.at[idxq_d.at[q]], ssem.at[b],
                             add=True)

        def stores_wait(kk, b, q):
            pltpu.make_async_copy(rl.at[b], ex_r.at[pl.ds(tbase + kk * C1, C1)],
                                  wsem.at[b]).wait()
            pltpu.make_async_copy(rl.at[b], s_sh.at[idxq_d.at[q]],
                                  ssem.at[b]).wait()

        def compute(b):
            @plsc.parallel_loop(0, C1, unroll=8)
            def row(i):
                v = rl[b, i, :] + rr[b, i, :]
                rl[b, i, :] = jnp.exp(jnp.maximum(v, 0.2 * v))

        for q in range(4):
            idx_load(q, q)
        idx_wait(0)
        gathers(0, 0)
        idx_wait(1)
        gathers(1, 1)

        def phase(k, j, p_is_dyn, p=None):
            b, q = j % 3, j % 6
            gathers_wait(b)
            compute(b)
            stores(k, b, q)
            jw = (j - 1) % 6
            if j >= 1:
                stores_wait(k - 1, jw % 3, jw % 6)
            elif p_is_dyn:
                @pl.when(p > 0)
                def _():
                    stores_wait(k - 1, jw % 3, jw % 6)
            j2 = (j + 2) % 6
            idx_wait(j2 % 6)
            gathers(j2 % 3, j2 % 6)
            idx_load(k + 4, (j + 4) % 6)

        def grp(p, _):
            k0 = p * 6
            for j in range(6):
                phase(k0 + j, j, True, p)
            return 0
        lax.fori_loop(0, M1, grp, 0)

        for k in range(M1 * 6, C1N):
            j = k % 6
            b, q = j % 3, j % 6
            gathers_wait(b)
            compute(b)
            stores(k, b, q)
            jw = (j - 1) % 6
            stores_wait(k - 1, jw % 3, jw % 6)
            if k + 2 < C1N:
                j2 = (j + 2) % 6
                idx_wait(j2 % 6)
                gathers(j2 % 3, j2 % 6)
            if k + 4 < C1N:
                idx_load(k + 4, (j + 4) % 6)
        jl = (C1N - 1) % 6
        stores_wait(C1N - 1, jl % 3, jl % 6)

    @pl.when(c == 0)
    def _():
        run(src_p, dst_p, tl_p, tr_p, ex_p)

    @pl.when(c == 1)
    def _():
        run(src_n, dst_n, tl_n, tr_n, ex_n)

    plsc.subcore_barrier()

    pltpu.sync_copy(s_sh.at[pl.ds(s * NPT, NPT)], fin_b)

    @plsc.parallel_loop(0, NPT, unroll=8)
    def fin(i):
        fin_b[i, :] = 1.0 / jnp.maximum(fin_b[i, :], 1e-16)

    @pl.when(c == 0)
    def _():
        pltpu.sync_copy(fin_b, inv_p.at[pl.ds(s * NPT, NPT)])

    @pl.when(c == 1)
    def _():
        pltpu.sync_copy(fin_b, inv_n.at[pl.ds(s * NPT, NPT)])


def _pass1(src_p, dst_p, src_n, dst_n, tl_p, tr_p, tl_n, tr_n, z16):
    f = pl.kernel(
        _pass1_body,
        out_type=[jax.ShapeDtypeStruct((E, L), jnp.float32),
                  jax.ShapeDtypeStruct((E, L), jnp.float32),
                  jax.ShapeDtypeStruct((NP, L), jnp.float32),
                  jax.ShapeDtypeStruct((NP, L), jnp.float32)],
        mesh=_sc_mesh(),
        compiler_params=pltpu.CompilerParams(use_tc_tiling_on_sc=False,
                                             needs_layout_passes=False),
        scratch_types=[
            pltpu.VMEM((6, C1), jnp.int32),
            pltpu.VMEM((6, C1), jnp.int32),
            pltpu.VMEM((3, C1, L), jnp.float32),
            pltpu.VMEM((3, C1, L), jnp.float32),
            pltpu.VMEM((NPT, L), jnp.float32),
            pltpu.VMEM_SHARED((NP, L), jnp.float32),
            pltpu.SemaphoreType.DMA((6,)),
            pltpu.SemaphoreType.DMA((3,)),
            pltpu.SemaphoreType.DMA((3,)),
            pltpu.SemaphoreType.DMA((3,)),
        ],
    )
    return f(src_p, dst_p, src_n, dst_n, tl_p, tr_p, tl_n, tr_n, z16)


NB = 3
NQ = 6
NCH = EPT // C2
GRP = 6
MAIN = (NCH - 5) // GRP


def _pass2_body(src_p, dst_p, src_n, dst_n, ex_p, ex_n, inv_p, inv_n,
                fp0, fp1, fn0, fn1, z128,
                op0, op1, on0, on1,
                idxq_s, idxq_d, exb, invb, fb, osh, isem, gsem, ssem):
    c = lax.axis_index("c")
    s = lax.axis_index("s")

    def run(src, dst, ex, inv, f_r, o_r, head_base):
        col = [jnp.full((L,), head_base + hh, jnp.int32) for hh in range(4)]
        tbase = s * EPT

        def idx_load(kk, q):
            pltpu.async_copy(src.at[pl.ds(tbase + kk * C2, C2)],
                             idxq_s.at[q], isem.at[q])
            pltpu.async_copy(dst.at[pl.ds(tbase + kk * C2, C2)],
                             idxq_d.at[q], isem.at[q])

        def idx_wait(q):
            pltpu.make_async_copy(src.at[pl.ds(0, C2)], idxq_s.at[q],
                                  isem.at[q]).wait()
            pltpu.make_async_copy(dst.at[pl.ds(0, C2)], idxq_d.at[q],
                                  isem.at[q]).wait()

        def gathers(kk, b, q):
            pltpu.async_copy(inv.at[idxq_d.at[q]], invb.at[b], gsem.at[b])
            pltpu.async_copy(f_r.at[idxq_s.at[q]], fb.at[b], gsem.at[b])
            pltpu.async_copy(ex.at[pl.ds(tbase + kk * C2, C2)], exb.at[b],
                             gsem.at[b])

        def gathers_wait(b):
            pltpu.make_async_copy(inv.at[pl.ds(0, C2)], invb.at[b],
                                  gsem.at[b]).wait()
            pltpu.make_async_copy(f_r.at[pl.ds(0, C2)], fb.at[b],
                                  gsem.at[b]).wait()
            pltpu.make_async_copy(ex.at[pl.ds(0, C2)], exb.at[b],
                                  gsem.at[b]).wait()

        def scatter(b, q):
            pltpu.async_copy(fb.at[b], osh.at[idxq_d.at[q]], ssem.at[b],
                             add=True)

        def scatter_wait(b, q):
            pltpu.make_async_copy(fb.at[b], osh.at[idxq_d.at[q]],
                                  ssem.at[b]).wait()

        def compute(b):
            @plsc.parallel_loop(0, C2, unroll=4)
            def row(i):
                al = exb[b, i, :] * invb[b, i, :]
                for hh in range(4):
                    bc = _bcast(al, col[hh])
                    for jj in range(2):
                        j = hh * 2 + jj
                        fv = fb[b, i, pl.ds(j * L, L)]
                        fb[b, i, pl.ds(j * L, L)] = fv * bc

        for q in range(4):
            idx_load(q, q)
        idx_wait(0)
        gathers(0, 0, 0)
        idx_wait(1)
        gathers(1, 1, 1)

        def phase(k, j, p_is_dyn, p=None):
            b, q = j % NB, j % NQ
            gathers_wait(b)
            compute(b)
            scatter(b, q)
            jw = (j - 1) % GRP
            if j >= 1:
                scatter_wait(jw % NB, jw % NQ)
            elif p_is_dyn:
                @pl.when(p > 0)
                def _():
                    scatter_wait(jw % NB, jw % NQ)
            j2 = (j + 2) % GRP
            idx_wait(j2 % NQ)
            gathers(k + 2, j2 % NB, j2 % NQ)
            idx_load(k + 4, (j + 4) % NQ)

        def grp(p, _):
            k0 = p * GRP
            for j in range(GRP):
                phase(k0 + j, j, True, p)
            return 0
        lax.fori_loop(0, MAIN, grp, 0)

        for k in range(MAIN * GRP, NCH):
            j = k % GRP
            b, q = j % NB, j % NQ
            gathers_wait(b)
            compute(b)
            scatter(b, q)
            jw = (j - 1) % GRP
            scatter_wait(jw % NB, jw % NQ)
            if k + 2 < NCH:
                j2 = (j + 2) % GRP
                idx_wait(j2 % NQ)
                gathers(k + 2, j2 % NB, j2 % NQ)
            if k + 4 < NCH:
                idx_load(k + 4, (j + 4) % NQ)
        jl = (NCH - 1) % GRP
        scatter_wait(jl % NB, jl % NQ)

        plsc.subcore_barrier()
        pltpu.sync_copy(osh.at[pl.ds(s * NPT, NPT)], o_r.at[pl.ds(s * NPT, NPT)])


    for (srcr, dstr, exr, invr, fh0, fh1, oh0, oh1) in (
            (src_p, dst_p, ex_p, inv_p, fp0, fp1, op0, op1),
            (src_n, dst_n, ex_n, inv_n, fn0, fn1, on0, on1)):
        pltpu.sync_copy(z128, osh.at[pl.ds(s * NPT, NPT)])
        plsc.subcore_barrier()

        @pl.when(c == 0)
        def _():
            run(srcr, dstr, exr, invr, fh0, oh0, 0)

        @pl.when(c == 1)
        def _():
            run(srcr, dstr, exr, invr, fh1, oh1, 4)

        plsc.subcore_barrier()


def _pass2(src_p, dst_p, src_n, dst_n, ex_p, ex_n, inv_p, inv_n,
           fp0, fp1, fn0, fn1, z128):
    f = pl.kernel(
        _pass2_body,
        out_type=[jax.ShapeDtypeStruct((NP, HALF), jnp.float32)] * 4,
        mesh=_sc_mesh(),
        compiler_params=pltpu.CompilerParams(use_tc_tiling_on_sc=False,
                                             needs_layout_passes=False),
        scratch_types=[
            pltpu.VMEM((NQ, C2), jnp.int32),
            pltpu.VMEM((NQ, C2), jnp.int32),
            pltpu.VMEM((NB, C2, L), jnp.float32),
            pltpu.VMEM((NB, C2, L), jnp.float32),
            pltpu.VMEM((NB, C2, HALF), jnp.float32),
            pltpu.VMEM_SHARED((NP, HALF), jnp.float32),
            pltpu.SemaphoreType.DMA((NQ,)),
            pltpu.SemaphoreType.DMA((NB,)),
            pltpu.SemaphoreType.DMA((NB,)),
        ],
    )
    return f(src_p, dst_p, src_n, dst_n, ex_p, ex_n, inv_p, inv_n,
             fp0, fp1, fn0, fn1, z128)


def _mlp_body(op0_ref, op1_ref, on0_ref, on1_ref, bp_ref, bn_ref,
              w1_ref, b1_ref, w2_ref, b2_ref,
              hp_ref, hn_ref, hf_ref):
    hp0 = op0_ref[...] + bp_ref[0:1, :HALF]
    hp1 = op1_ref[...] + bp_ref[0:1, HALF:]
    hn0 = on0_ref[...] + bn_ref[0:1, :HALF]
    hn1 = on1_ref[...] + bn_ref[0:1, HALF:]
    hp_ref[:, :HALF] = hp0
    hp_ref[:, HALF:] = hp1
    hn_ref[:, :HALF] = hn0
    hn_ref[:, HALF:] = hn1
    w1 = w1_ref[...]
    z = (jnp.dot(hp0, w1[0:HALF, :], preferred_element_type=jnp.float32)
         + jnp.dot(hp1, w1[HALF:HID, :], preferred_element_type=jnp.float32)
         + jnp.dot(hn0, w1[HID:HID + HALF, :], preferred_element_type=jnp.float32)
         + jnp.dot(hn1, w1[HID + HALF:, :], preferred_element_type=jnp.float32)
         + b1_ref[0:1, :])
    z = jnp.maximum(z, 0.0)
    hf_ref[...] = jnp.dot(z, w2_ref[...], preferred_element_type=jnp.float32) + b2_ref[0:1, :]


def _mlp(op0, op1, on0, on1, bp, bn, w1, b1, w2, b2):
    R = 1000
    grid = (N // R,)
    bs_h = pl.BlockSpec((R, HALF), lambda i: (i, 0))
    return pl.pallas_call(
        _mlp_body,
        grid=grid,
        in_specs=[bs_h, bs_h, bs_h, bs_h,
                  pl.BlockSpec((1, HID), lambda i: (0, 0)),
                  pl.BlockSpec((1, HID), lambda i: (0, 0)),
                  pl.BlockSpec((2 * HID, HID), lambda i: (0, 0)),
                  pl.BlockSpec((1, HID), lambda i: (0, 0)),
                  pl.BlockSpec((HID, OUT_DIM), lambda i: (0, 0)),
                  pl.BlockSpec((1, OUT_DIM), lambda i: (0, 0))],
        out_specs=[pl.BlockSpec((R, HID), lambda i: (i, 0)),
                   pl.BlockSpec((R, HID), lambda i: (i, 0)),
                   pl.BlockSpec((R, OUT_DIM), lambda i: (i, 0))],
        out_shape=[jax.ShapeDtypeStruct((N, HID), jnp.float32),
                   jax.ShapeDtypeStruct((N, HID), jnp.float32),
                   jax.ShapeDtypeStruct((N, OUT_DIM), jnp.float32)],
    )(op0, op1, on0, on1, bp, bn, w1, b1, w2, b2)


def kernel(features, pos_edge_index, neg_edge_index, W_pos, attn_l_pos,
           attn_r_pos, b_pos, W_neg, attn_l_neg, attn_r_neg, b_neg,
           W1, b1, W2, b2):
    src_p = pos_edge_index[0].astype(jnp.int32)
    dst_p = pos_edge_index[1].astype(jnp.int32)
    src_n = neg_edge_index[0].astype(jnp.int32)
    dst_n = neg_edge_index[1].astype(jnp.int32)

    eye = jnp.eye(H, L, dtype=jnp.float32)
    alp = (attn_l_pos[:, :, None] * eye[:, None, :]).reshape(HID, L)
    arp = (attn_r_pos[:, :, None] * eye[:, None, :]).reshape(HID, L)
    aln = (attn_l_neg[:, :, None] * eye[:, None, :]).reshape(HID, L)
    arn = (attn_r_neg[:, :, None] * eye[:, None, :]).reshape(HID, L)

    z16 = jnp.zeros((NPT, L), jnp.float32)
    z128 = jnp.zeros((NPT, HALF), jnp.float32)

    fp0, fp1, fn0, fn1, tlp, trp, tln, trn = _encode(
        features, W_pos, W_neg, alp, arp, aln, arn)

    ex_p, ex_n, inv_p, inv_n = _pass1(
        src_p, dst_p, src_n, dst_n, tlp, trp, tln, trn, z16)

    op0, op1, on0, on1 = _pass2(src_p, dst_p, src_n, dst_n, ex_p, ex_n,
                                inv_p, inv_n, fp0, fp1, fn0, fn1, z128)

    h_pos, h_neg, h_final = _mlp(
        op0, op1, on0, on1,
        b_pos.reshape(1, HID), b_neg.reshape(1, HID),
        W1, b1.reshape(1, HID), W2, b2.reshape(1, OUT_DIM))
    return (h_pos, h_neg, h_final)


# speedup vs baseline: 108.0779x; 1.1113x over previous
import functools

import jax
import jax.numpy as jnp
from jax import lax
from jax.experimental import pallas as pl
from jax.experimental.pallas import tpu as pltpu
from jax.experimental.pallas import tpu_sc as plsc

N = 10000
IN_DIM = 128
HID = 256
OUT_DIM = 128
H = 8
DH = HID // H
E = 160000

NC = 2
NS = 16
L = 16

NP = 10240
NPT = NP // NS
EPT = E // NS
C1 = 400
C2 = 80
HALF = HID // 2


def _sc_mesh():
    return plsc.VectorSubcoreMesh(core_axis_name="c", subcore_axis_name="s",
                                  num_cores=NC, num_subcores=NS)


_BCAST_DNUMS = lax.GatherDimensionNumbers(
    offset_dims=(), collapsed_slice_dims=(0,), start_index_map=(0,))


def _bcast(v, idx):
    return lax.gather(v, idx[:, None], _BCAST_DNUMS, (1,),
                      mode=lax.GatherScatterMode.PROMISE_IN_BOUNDS)


def _encode_body(x_ref, wp_ref, wn_ref, alp_ref, arp_ref, aln_ref, arn_ref,
                 fp0_ref, fp1_ref, fn0_ref, fn1_ref,
                 tlp_ref, trp_ref, tln_ref, trn_ref):
    x = x_ref[...]
    fp = jnp.dot(x, wp_ref[...], preferred_element_type=jnp.float32)
    fn = jnp.dot(x, wn_ref[...], preferred_element_type=jnp.float32)
    fp0_ref[...] = fp[:, :HALF].astype(jnp.bfloat16)
    fp1_ref[...] = fp[:, HALF:].astype(jnp.bfloat16)
    fn0_ref[...] = fn[:, :HALF].astype(jnp.bfloat16)
    fn1_ref[...] = fn[:, HALF:].astype(jnp.bfloat16)
    tlp_ref[...] = jnp.dot(fp, alp_ref[...], preferred_element_type=jnp.float32)
    trp_ref[...] = jnp.dot(fp, arp_ref[...], preferred_element_type=jnp.float32)
    tln_ref[...] = jnp.dot(fn, aln_ref[...], preferred_element_type=jnp.float32)
    trn_ref[...] = jnp.dot(fn, arn_ref[...], preferred_element_type=jnp.float32)


def _encode(x, wp, wn, alp, arp, aln, arn):
    R = 1000
    grid = (N // R,)
    bs_x = pl.BlockSpec((R, IN_DIM), lambda i: (i, 0))
    bs_w = pl.BlockSpec((IN_DIM, HID), lambda i: (0, 0))
    bs_a = pl.BlockSpec((HID, L), lambda i: (0, 0))
    bs_f = pl.BlockSpec((R, HALF), lambda i: (i, 0))
    bs_t = pl.BlockSpec((R, L), lambda i: (i, 0))
    outs = [jax.ShapeDtypeStruct((N, HALF), jnp.bfloat16)] * 4 + \
           [jax.ShapeDtypeStruct((N, L), jnp.float32)] * 4
    return pl.pallas_call(
        _encode_body,
        grid=grid,
        in_specs=[bs_x, bs_w, bs_w, bs_a, bs_a, bs_a, bs_a],
        out_specs=[bs_f] * 4 + [bs_t] * 4,
        out_shape=outs,
    )(x, wp, wn, alp, arp, aln, arn)


C1N = EPT // C1
M1 = (C1N - 5) // 6


def _pass1_body(src_p, dst_p, src_n, dst_n, tl_p, tr_p, tl_n, tr_n, z16,
                ex_p, ex_n, inv_p, inv_n,
                idxq_s, idxq_d, rl, rr, fin_b, s_sh, isem, gsem, wsem, ssem):
    c = lax.axis_index("c")
    s = lax.axis_index("s")

    pltpu.sync_copy(z16, s_sh.at[pl.ds(s * NPT, NPT)])
    plsc.subcore_barrier()

    def run(src_r, dst_r, tl_r, tr_r, ex_r):
        tbase = s * EPT

        def idx_load(kk, q):
            pltpu.async_copy(src_r.at[pl.ds(tbase + kk * C1, C1)],
                             idxq_s.at[q], isem.at[q])
            pltpu.async_copy(dst_r.at[pl.ds(tbase + kk * C1, C1)],
                             idxq_d.at[q], isem.at[q])

        def idx_wait(q):
            pltpu.make_async_copy(src_r.at[pl.ds(0, C1)], idxq_s.at[q],
                                  isem.at[q]).wait()
            pltpu.make_async_copy(dst_r.at[pl.ds(0, C1)], idxq_d.at[q],
                                  isem.at[q]).wait()

        def gathers(b, q):
            pltpu.async_copy(tl_r.at[idxq_s.at[q]], rl.at[b], gsem.at[b])
            pltpu.async_copy(tr_r.at[idxq_d.at[q]], rr.at[b], gsem.at[b])

        def gathers_wait(b):
            pltpu.make_async_copy(tl_r.at[pl.ds(0, C1)], rl.at[b],
                                  gsem.at[b]).wait()
            pltpu.make_async_copy(tr_r.at[pl.ds(0, C1)], rr.at[b],
                                  gsem.at[b]).wait()

        def stores(kk, b, q):
            pltpu.async_copy(rl.at[b], ex_r.at[pl.ds(tbase + kk * C1, C1)],
                             wsem.at[b])
            pltpu.async_copy(rl.at[b], s_sh.at[idxq_d.at[q]], ssem.at[b],
                             add=True)

        def stores_wait(kk, b, q):
            pltpu.make_async_copy(rl.at[b], ex_r.at[pl.ds(tbase + kk * C1, C1)],
                                  wsem.at[b]).wait()
            pltpu.make_async_copy(rl.at[b], s_sh.at[idxq_d.at[q]],
                                  ssem.at[b]).wait()

        def compute(b):
            @plsc.parallel_loop(0, C1, unroll=8)
            def row(i):
                v = rl[b, i, :] + rr[b, i, :]
                rl[b, i, :] = jnp.exp(jnp.maximum(v, 0.2 * v))

        for q in range(4):
            idx_load(q, q)
        idx_wait(0)
        gathers(0, 0)
        idx_wait(1)
        gathers(1, 1)

        def phase(k, j, p_is_dyn, p=None):
            b, q = j % 3, j % 6
            gathers_wait(b)
            compute(b)
            stores(k, b, q)
            jw = (j - 1) % 6
            if j >= 1:
                stores_wait(k - 1, jw % 3, jw % 6)
            elif p_is_dyn:
                @pl.when(p > 0)
                def _():
                    stores_wait(k - 1, jw % 3, jw % 6)
            j2 = (j + 2) % 6
            idx_wait(j2 % 6)
            gathers(j2 % 3, j2 % 6)
            idx_load(k + 4, (j + 4) % 6)

        def grp(p, _):
            k0 = p * 6
            for j in range(6):
                phase(k0 + j, j, True, p)
            return 0
        lax.fori_loop(0, M1, grp, 0)

        for k in range(M1 * 6, C1N):
            j = k % 6
            b, q = j % 3, j % 6
            gathers_wait(b)
            compute(b)
            stores(k, b, q)
            jw = (j - 1) % 6
            stores_wait(k - 1, jw % 3, jw % 6)
            if k + 2 < C1N:
                j2 = (j + 2) % 6
                idx_wait(j2 % 6)
                gathers(j2 % 3, j2 % 6)
            if k + 4 < C1N:
                idx_load(k + 4, (j + 4) % 6)
        jl = (C1N - 1) % 6
        stores_wait(C1N - 1, jl % 3, jl % 6)

    @pl.when(c == 0)
    def _():
        run(src_p, dst_p, tl_p, tr_p, ex_p)

    @pl.when(c == 1)
    def _():
        run(src_n, dst_n, tl_n, tr_n, ex_n)

    plsc.subcore_barrier()

    pltpu.sync_copy(s_sh.at[pl.ds(s * NPT, NPT)], fin_b)

    @plsc.parallel_loop(0, NPT, unroll=8)
    def fin(i):
        fin_b[i, :] = 1.0 / jnp.maximum(fin_b[i, :], 1e-16)

    @pl.when(c == 0)
    def _():
        pltpu.sync_copy(fin_b, inv_p.at[pl.ds(s * NPT, NPT)])

    @pl.when(c == 1)
    def _():
        pltpu.sync_copy(fin_b, inv_n.at[pl.ds(s * NPT, NPT)])


def _pass1(src_p, dst_p, src_n, dst_n, tl_p, tr_p, tl_n, tr_n, z16):
    f = pl.kernel(
        _pass1_body,
        out_type=[jax.ShapeDtypeStruct((E, L), jnp.float32),
                  jax.ShapeDtypeStruct((E, L), jnp.float32),
                  jax.ShapeDtypeStruct((NP, L), jnp.float32),
                  jax.ShapeDtypeStruct((NP, L), jnp.float32)],
        mesh=_sc_mesh(),
        compiler_params=pltpu.CompilerParams(use_tc_tiling_on_sc=False,
                                             needs_layout_passes=False),
        scratch_types=[
            pltpu.VMEM((6, C1), jnp.int32),
            pltpu.VMEM((6, C1), jnp.int32),
            pltpu.VMEM((3, C1, L), jnp.float32),
            pltpu.VMEM((3, C1, L), jnp.float32),
            pltpu.VMEM((NPT, L), jnp.float32),
            pltpu.VMEM_SHARED((NP, L), jnp.float32),
            pltpu.SemaphoreType.DMA((6,)),
            pltpu.SemaphoreType.DMA((3,)),
            pltpu.SemaphoreType.DMA((3,)),
            pltpu.SemaphoreType.DMA((3,)),
        ],
    )
    return f(src_p, dst_p, src_n, dst_n, tl_p, tr_p, tl_n, tr_n, z16)


NB = 3
NQ = 6
NCH = EPT // C2
GRP = 6
MAIN = (NCH - 5) // GRP


def _pass2_body(src_p, dst_p, src_n, dst_n, ex_p, ex_n, inv_p, inv_n,
                fp0, fp1, fn0, fn1, z128,
                op0, op1, on0, on1,
                idxq_s, idxq_d, exb, invb, fbh, prod, osh, isem, gsem, ssem):
    c = lax.axis_index("c")
    s = lax.axis_index("s")

    def run(src, dst, ex, inv, f_r, o_r, head_base):
        col = [jnp.full((L,), head_base + hh, jnp.int32) for hh in range(4)]
        tbase = s * EPT

        def idx_load(kk, q):
            pltpu.async_copy(src.at[pl.ds(tbase + kk * C2, C2)],
                             idxq_s.at[q], isem.at[q])
            pltpu.async_copy(dst.at[pl.ds(tbase + kk * C2, C2)],
                             idxq_d.at[q], isem.at[q])

        def idx_wait(q):
            pltpu.make_async_copy(src.at[pl.ds(0, C2)], idxq_s.at[q],
                                  isem.at[q]).wait()
            pltpu.make_async_copy(dst.at[pl.ds(0, C2)], idxq_d.at[q],
                                  isem.at[q]).wait()

        def gathers(kk, b, q):
            pltpu.async_copy(inv.at[idxq_d.at[q]], invb.at[b], gsem.at[b])
            pltpu.async_copy(f_r.at[idxq_s.at[q]], fbh.at[b], gsem.at[b])
            pltpu.async_copy(ex.at[pl.ds(tbase + kk * C2, C2)], exb.at[b],
                             gsem.at[b])

        def gathers_wait(b):
            pltpu.make_async_copy(inv.at[pl.ds(0, C2)], invb.at[b],
                                  gsem.at[b]).wait()
            pltpu.make_async_copy(f_r.at[pl.ds(0, C2)], fbh.at[b],
                                  gsem.at[b]).wait()
            pltpu.make_async_copy(ex.at[pl.ds(0, C2)], exb.at[b],
                                  gsem.at[b]).wait()

        def scatter(pb, q):
            pltpu.async_copy(prod.at[pb], osh.at[idxq_d.at[q]], ssem.at[pb],
                             add=True)

        def scatter_wait(pb, q):
            pltpu.make_async_copy(prod.at[pb], osh.at[idxq_d.at[q]],
                                  ssem.at[pb]).wait()

        def compute(b, pb):
            @plsc.parallel_loop(0, C2, unroll=4)
            def row(i):
                al = exb[b, i, :] * invb[b, i, :]
                for hh in range(4):
                    bc = _bcast(al, col[hh])
                    v32 = fbh[b, i, pl.ds(hh * 32, 32)]
                    ev, od = plsc.unpack(v32, format=plsc.PackFormat.INTERLEAVED)
                    prod[pb, i, pl.ds(hh * 32, L)] = ev * bc
                    prod[pb, i, pl.ds(hh * 32 + L, L)] = od * bc

        for q in range(4):
            idx_load(q, q)
        idx_wait(0)
        gathers(0, 0, 0)
        idx_wait(1)
        gathers(1, 1, 1)

        def phase(k, j, p_is_dyn, p=None):
            b, q, pb = j % NB, j % NQ, j % 2
            gathers_wait(b)
            compute(b, pb)
            scatter(pb, q)
            jw = (j - 1) % GRP
            if j >= 1:
                scatter_wait(jw % 2, jw % NQ)
            elif p_is_dyn:
                @pl.when(p > 0)
                def _():
                    scatter_wait(jw % 2, jw % NQ)
            j2 = (j + 2) % GRP
            idx_wait(j2 % NQ)
            gathers(k + 2, j2 % NB, j2 % NQ)
            idx_load(k + 4, (j + 4) % NQ)

        def grp(p, _):
            k0 = p * GRP
            for j in range(GRP):
                phase(k0 + j, j, True, p)
            return 0
        lax.fori_loop(0, MAIN, grp, 0)

        for k in range(MAIN * GRP, NCH):
            j = k % GRP
            b, q, pb = j % NB, j % NQ, j % 2
            gathers_wait(b)
            compute(b, pb)
            scatter(pb, q)
            jw = (j - 1) % GRP
            scatter_wait(jw % 2, jw % NQ)
            if k + 2 < NCH:
                j2 = (j + 2) % GRP
                idx_wait(j2 % NQ)
                gathers(k + 2, j2 % NB, j2 % NQ)
            if k + 4 < NCH:
                idx_load(k + 4, (j + 4) % NQ)
        jl = (NCH - 1) % GRP
        scatter_wait(jl % 2, jl % NQ)

        plsc.subcore_barrier()
        pltpu.sync_copy(osh.at[pl.ds(s * NPT, NPT)], o_r.at[pl.ds(s * NPT, NPT)])


    for (srcr, dstr, exr, invr, fh0, fh1, oh0, oh1) in (
            (src_p, dst_p, ex_p, inv_p, fp0, fp1, op0, op1),
            (src_n, dst_n, ex_n, inv_n, fn0, fn1, on0, on1)):
        pltpu.sync_copy(z128, osh.at[pl.ds(s * NPT, NPT)])
        plsc.subcore_barrier()

        @pl.when(c == 0)
        def _():
            run(srcr, dstr, exr, invr, fh0, oh0, 0)

        @pl.when(c == 1)
        def _():
            run(srcr, dstr, exr, invr, fh1, oh1, 4)

        plsc.subcore_barrier()


def _pass2(src_p, dst_p, src_n, dst_n, ex_p, ex_n, inv_p, inv_n,
           fp0, fp1, fn0, fn1, z128):
    f = pl.kernel(
        _pass2_body,
        out_type=[jax.ShapeDtypeStruct((NP, HALF), jnp.float32)] * 4,
        mesh=_sc_mesh(),
        compiler_params=pltpu.CompilerParams(use_tc_tiling_on_sc=False,
                                             needs_layout_passes=False),
        scratch_types=[
            pltpu.VMEM((NQ, C2), jnp.int32),
            pltpu.VMEM((NQ, C2), jnp.int32),
            pltpu.VMEM((NB, C2, L), jnp.float32),
            pltpu.VMEM((NB, C2, L), jnp.float32),
            pltpu.VMEM((NB, C2, HALF), jnp.bfloat16),
            pltpu.VMEM((2, C2, HALF), jnp.float32),
            pltpu.VMEM_SHARED((NP, HALF), jnp.float32),
            pltpu.SemaphoreType.DMA((NQ,)),
            pltpu.SemaphoreType.DMA((NB,)),
            pltpu.SemaphoreType.DMA((2,)),
        ],
    )
    return f(src_p, dst_p, src_n, dst_n, ex_p, ex_n, inv_p, inv_n,
             fp0, fp1, fn0, fn1, z128)


def _mlp_body(op0_ref, op1_ref, on0_ref, on1_ref, m0_ref, m1_ref,
              bp_ref, bn_ref, w1_ref, b1_ref, w2_ref, b2_ref,
              hp_ref, hn_ref, hf_ref):
    m0 = m0_ref[...]
    m1 = m1_ref[...]
    hp = (jnp.dot(op0_ref[...], m0, preferred_element_type=jnp.float32)
          + jnp.dot(op1_ref[...], m1, preferred_element_type=jnp.float32)
          + bp_ref[0:1, :])
    hn = (jnp.dot(on0_ref[...], m0, preferred_element_type=jnp.float32)
          + jnp.dot(on1_ref[...], m1, preferred_element_type=jnp.float32)
          + bn_ref[0:1, :])
    hp_ref[...] = hp
    hn_ref[...] = hn
    w1 = w1_ref[...]
    z = (jnp.dot(hp, w1[0:HID, :], preferred_element_type=jnp.float32)
         + jnp.dot(hn, w1[HID:, :], preferred_element_type=jnp.float32)
         + b1_ref[0:1, :])
    z = jnp.maximum(z, 0.0)
    hf_ref[...] = jnp.dot(z, w2_ref[...], preferred_element_type=jnp.float32) + b2_ref[0:1, :]


def _mlp(op0, op1, on0, on1, m0, m1, bp, bn, w1, b1, w2, b2):
    R = 1000
    grid = (N // R,)
    bs_h = pl.BlockSpec((R, HALF), lambda i: (i, 0))
    bs_m = pl.BlockSpec((HALF, HID), lambda i: (0, 0))
    return pl.pallas_call(
        _mlp_body,
        grid=grid,
        in_specs=[bs_h, bs_h, bs_h, bs_h, bs_m, bs_m,
                  pl.BlockSpec((1, HID), lambda i: (0, 0)),
                  pl.BlockSpec((1, HID), lambda i: (0, 0)),
                  pl.BlockSpec((2 * HID, HID), lambda i: (0, 0)),
                  pl.BlockSpec((1, HID), lambda i: (0, 0)),
                  pl.BlockSpec((HID, OUT_DIM), lambda i: (0, 0)),
                  pl.BlockSpec((1, OUT_DIM), lambda i: (0, 0))],
        out_specs=[pl.BlockSpec((R, HID), lambda i: (i, 0)),
                   pl.BlockSpec((R, HID), lambda i: (i, 0)),
                   pl.BlockSpec((R, OUT_DIM), lambda i: (i, 0))],
        out_shape=[jax.ShapeDtypeStruct((N, HID), jnp.float32),
                   jax.ShapeDtypeStruct((N, HID), jnp.float32),
                   jax.ShapeDtypeStruct((N, OUT_DIM), jnp.float32)],
    )(op0, op1, on0, on1, m0, m1, bp, bn, w1, b1, w2, b2)


def kernel(features, pos_edge_index, neg_edge_index, W_pos, attn_l_pos,
           attn_r_pos, b_pos, W_neg, attn_l_neg, attn_r_neg, b_neg,
           W1, b1, W2, b2):
    src_p = pos_edge_index[0].astype(jnp.int32)
    dst_p = pos_edge_index[1].astype(jnp.int32)
    src_n = neg_edge_index[0].astype(jnp.int32)
    dst_n = neg_edge_index[1].astype(jnp.int32)

    eye = jnp.eye(H, L, dtype=jnp.float32)
    alp = (attn_l_pos[:, :, None] * eye[:, None, :]).reshape(HID, L)
    arp = (attn_r_pos[:, :, None] * eye[:, None, :]).reshape(HID, L)
    aln = (attn_l_neg[:, :, None] * eye[:, None, :]).reshape(HID, L)
    arn = (attn_r_neg[:, :, None] * eye[:, None, :]).reshape(HID, L)

    z16 = jnp.zeros((NPT, L), jnp.float32)
    z128 = jnp.zeros((NPT, HALF), jnp.float32)

    fp0, fp1, fn0, fn1, tlp, trp, tln, trn = _encode(
        features, W_pos, W_neg, alp, arp, aln, arn)

    ex_p, ex_n, inv_p, inv_n = _pass1(
        src_p, dst_p, src_n, dst_n, tlp, trp, tln, trn, z16)

    op0, op1, on0, on1 = _pass2(src_p, dst_p, src_n, dst_n, ex_p, ex_n,
                                inv_p, inv_n, fp0, fp1, fn0, fn1, z128)

    ow = [32 * (j // 32) + (2 * (j % 32) if (j % 32) < L else
                            2 * ((j % 32) - L) + 1) for j in range(HALF)]
    eye_h = jnp.eye(HID, dtype=jnp.float32)
    m0 = eye_h[jnp.array(ow)]
    m1 = eye_h[jnp.array([o + HALF for o in ow])]

    h_pos, h_neg, h_final = _mlp(
        op0, op1, on0, on1, m0, m1,
        b_pos.reshape(1, HID), b_neg.reshape(1, HID),
        W1, b1.reshape(1, HID), W2, b2.reshape(1, OUT_DIM))
    return (h_pos, h_neg, h_final)


# speedup vs baseline: 109.5916x; 1.0140x over previous
import functools

import jax
import jax.numpy as jnp
from jax import lax
from jax.experimental import pallas as pl
from jax.experimental.pallas import tpu as pltpu
from jax.experimental.pallas import tpu_sc as plsc

N = 10000
IN_DIM = 128
HID = 256
OUT_DIM = 128
H = 8
DH = HID // H
E = 160000

NC = 2
NS = 16
L = 16

NP = 10240
NPT = NP // NS
EPT = E // NS
C1 = 400
C2 = 80
HALF = HID // 2


def _sc_mesh():
    return plsc.VectorSubcoreMesh(core_axis_name="c", subcore_axis_name="s",
                                  num_cores=NC, num_subcores=NS)


_BCAST_DNUMS = lax.GatherDimensionNumbers(
    offset_dims=(), collapsed_slice_dims=(0,), start_index_map=(0,))


def _bcast(v, idx):
    return lax.gather(v, idx[:, None], _BCAST_DNUMS, (1,),
                      mode=lax.GatherScatterMode.PROMISE_IN_BOUNDS)


def _encode_body(x_ref, wp_ref, wn_ref, alp_ref, arp_ref, aln_ref, arn_ref,
                 fp0_ref, fp1_ref, fn0_ref, fn1_ref,
                 tlp_ref, trp_ref, tln_ref, trn_ref):
    x = x_ref[...]
    fp = jnp.dot(x, wp_ref[...], preferred_element_type=jnp.float32)
    fn = jnp.dot(x, wn_ref[...], preferred_element_type=jnp.float32)
    fp0_ref[...] = fp[:, :HALF].astype(jnp.bfloat16)
    fp1_ref[...] = fp[:, HALF:].astype(jnp.bfloat16)
    fn0_ref[...] = fn[:, :HALF].astype(jnp.bfloat16)
    fn1_ref[...] = fn[:, HALF:].astype(jnp.bfloat16)
    tlp_ref[...] = jnp.dot(fp, alp_ref[...], preferred_element_type=jnp.float32)
    trp_ref[...] = jnp.dot(fp, arp_ref[...], preferred_element_type=jnp.float32)
    tln_ref[...] = jnp.dot(fn, aln_ref[...], preferred_element_type=jnp.float32)
    trn_ref[...] = jnp.dot(fn, arn_ref[...], preferred_element_type=jnp.float32)


def _encode(x, wp, wn, alp, arp, aln, arn):
    R = 2000
    grid = (N // R,)
    bs_x = pl.BlockSpec((R, IN_DIM), lambda i: (i, 0))
    bs_w = pl.BlockSpec((IN_DIM, HID), lambda i: (0, 0))
    bs_a = pl.BlockSpec((HID, L), lambda i: (0, 0))
    bs_f = pl.BlockSpec((R, HALF), lambda i: (i, 0))
    bs_t = pl.BlockSpec((R, L), lambda i: (i, 0))
    outs = [jax.ShapeDtypeStruct((N, HALF), jnp.bfloat16)] * 4 + \
           [jax.ShapeDtypeStruct((N, L), jnp.float32)] * 4
    return pl.pallas_call(
        _encode_body,
        grid=grid,
        in_specs=[bs_x, bs_w, bs_w, bs_a, bs_a, bs_a, bs_a],
        out_specs=[bs_f] * 4 + [bs_t] * 4,
        out_shape=outs,
    )(x, wp, wn, alp, arp, aln, arn)


C1N = EPT // C1
M1 = (C1N - 5) // 6


def _pass1_body(src_p, dst_p, src_n, dst_n, tl_p, tr_p, tl_n, tr_n, z16,
                ex_p, ex_n, inv_p, inv_n,
                idxq_s, idxq_d, rl, rr, fin_b, s_sh, isem, gsem, wsem, ssem):
    c = lax.axis_index("c")
    s = lax.axis_index("s")

    pltpu.sync_copy(z16, s_sh.at[pl.ds(s * NPT, NPT)])
    plsc.subcore_barrier()

    def run(src_r, dst_r, tl_r, tr_r, ex_r):
        tbase = s * EPT

        def idx_load(kk, q):
            pltpu.async_copy(src_r.at[pl.ds(tbase + kk * C1, C1)],
                             idxq_s.at[q], isem.at[q])
            pltpu.async_copy(dst_r.at[pl.ds(tbase + kk * C1, C1)],
                             idxq_d.at[q], isem.at[q])

        def idx_wait(q):
            pltpu.make_async_copy(src_r.at[pl.ds(0, C1)], idxq_s.at[q],
                                  isem.at[q]).wait()
            pltpu.make_async_copy(dst_r.at[pl.ds(0, C1)], idxq_d.at[q],
                                  isem.at[q]).wait()

        def gathers(b, q):
            pltpu.async_copy(tl_r.at[idxq_s.at[q]], rl.at[b], gsem.at[b])
            pltpu.async_copy(tr_r.at[idxq_d.at[q]], rr.at[b], gsem.at[b])

        def gathers_wait(b):
            pltpu.make_async_copy(tl_r.at[pl.ds(0, C1)], rl.at[b],
                                  gsem.at[b]).wait()
            pltpu.make_async_copy(tr_r.at[pl.ds(0, C1)], rr.at[b],
                                  gsem.at[b]).wait()

        def stores(kk, b, q):
            pltpu.async_copy(rl.at[b], ex_r.at[pl.ds(tbase + kk * C1, C1)],
                             wsem.at[b])
            pltpu.async_copy(rl.at[b], s_sh.at[idxq_d.at[q]], ssem.at[b],
                             add=True)

        def stores_wait(kk, b, q):
            pltpu.make_async_copy(rl.at[b], ex_r.at[pl.ds(tbase + kk * C1, C1)],
                                  wsem.at[b]).wait()
            pltpu.make_async_copy(rl.at[b], s_sh.at[idxq_d.at[q]],
                                  ssem.at[b]).wait()

        def compute(b):
            @plsc.parallel_loop(0, C1, unroll=8)
            def row(i):
                v = rl[b, i, :] + rr[b, i, :]
                rl[b, i, :] = jnp.exp(jnp.maximum(v, 0.2 * v))

        for q in range(4):
            idx_load(q, q)
        idx_wait(0)
        gathers(0, 0)
        idx_wait(1)
        gathers(1, 1)

        def phase(k, j, p_is_dyn, p=None):
            b, q = j % 3, j % 6
            gathers_wait(b)
            compute(b)
            stores(k, b, q)
            jw = (j - 1) % 6
            if j >= 1:
                stores_wait(k - 1, jw % 3, jw % 6)
            elif p_is_dyn:
                @pl.when(p > 0)
                def _():
                    stores_wait(k - 1, jw % 3, jw % 6)
            j2 = (j + 2) % 6
            idx_wait(j2 % 6)
            gathers(j2 % 3, j2 % 6)
            idx_load(k + 4, (j + 4) % 6)

        def grp(p, _):
            k0 = p * 6
            for j in range(6):
                phase(k0 + j, j, True, p)
            return 0
        lax.fori_loop(0, M1, grp, 0)

        for k in range(M1 * 6, C1N):
            j = k % 6
            b, q = j % 3, j % 6
            gathers_wait(b)
            compute(b)
            stores(k, b, q)
            jw = (j - 1) % 6
            stores_wait(k - 1, jw % 3, jw % 6)
            if k + 2 < C1N:
                j2 = (j + 2) % 6
                idx_wait(j2 % 6)
                gathers(j2 % 3, j2 % 6)
            if k + 4 < C1N:
                idx_load(k + 4, (j + 4) % 6)
        jl = (C1N - 1) % 6
        stores_wait(C1N - 1, jl % 3, jl % 6)

    @pl.when(c == 0)
    def _():
        run(src_p, dst_p, tl_p, tr_p, ex_p)

    @pl.when(c == 1)
    def _():
        run(src_n, dst_n, tl_n, tr_n, ex_n)

    plsc.subcore_barrier()

    pltpu.sync_copy(s_sh.at[pl.ds(s * NPT, NPT)], fin_b)

    @plsc.parallel_loop(0, NPT, unroll=8)
    def fin(i):
        fin_b[i, :] = 1.0 / jnp.maximum(fin_b[i, :], 1e-16)

    @pl.when(c == 0)
    def _():
        pltpu.sync_copy(fin_b, inv_p.at[pl.ds(s * NPT, NPT)])

    @pl.when(c == 1)
    def _():
        pltpu.sync_copy(fin_b, inv_n.at[pl.ds(s * NPT, NPT)])


def _pass1(src_p, dst_p, src_n, dst_n, tl_p, tr_p, tl_n, tr_n, z16):
    f = pl.kernel(
        _pass1_body,
        out_type=[jax.ShapeDtypeStruct((E, L), jnp.float32),
                  jax.ShapeDtypeStruct((E, L), jnp.float32),
                  jax.ShapeDtypeStruct((NP, L), jnp.float32),
                  jax.ShapeDtypeStruct((NP, L), jnp.float32)],
        mesh=_sc_mesh(),
        compiler_params=pltpu.CompilerParams(use_tc_tiling_on_sc=False,
                                             needs_layout_passes=False),
        scratch_types=[
            pltpu.VMEM((6, C1), jnp.int32),
            pltpu.VMEM((6, C1), jnp.int32),
            pltpu.VMEM((3, C1, L), jnp.float32),
            pltpu.VMEM((3, C1, L), jnp.float32),
            pltpu.VMEM((NPT, L), jnp.float32),
            pltpu.VMEM_SHARED((NP, L), jnp.float32),
            pltpu.SemaphoreType.DMA((6,)),
            pltpu.SemaphoreType.DMA((3,)),
            pltpu.SemaphoreType.DMA((3,)),
            pltpu.SemaphoreType.DMA((3,)),
        ],
    )
    return f(src_p, dst_p, src_n, dst_n, tl_p, tr_p, tl_n, tr_n, z16)


NB = 3
NQ = 6
NCH = EPT // C2
GRP = 6
MAIN = (NCH - 5) // GRP


def _pass2_body(src_p, dst_p, src_n, dst_n, ex_p, ex_n, inv_p, inv_n,
                fp0, fp1, fn0, fn1, z128,
                op0, op1, on0, on1,
                idxq_s, idxq_d, exb, invb, fbh, prod, osh, isem, gsem, ssem):
    c = lax.axis_index("c")
    s = lax.axis_index("s")

    def run(src, dst, ex, inv, f_r, o_r, head_base):
        col = [jnp.full((L,), head_base + hh, jnp.int32) for hh in range(4)]
        tbase = s * EPT

        def idx_load(kk, q):
            pltpu.async_copy(src.at[pl.ds(tbase + kk * C2, C2)],
                             idxq_s.at[q], isem.at[q])
            pltpu.async_copy(dst.at[pl.ds(tbase + kk * C2, C2)],
                             idxq_d.at[q], isem.at[q])

        def idx_wait(q):
            pltpu.make_async_copy(src.at[pl.ds(0, C2)], idxq_s.at[q],
                                  isem.at[q]).wait()
            pltpu.make_async_copy(dst.at[pl.ds(0, C2)], idxq_d.at[q],
                                  isem.at[q]).wait()

        def gathers(kk, b, q):
            pltpu.async_copy(inv.at[idxq_d.at[q]], invb.at[b], gsem.at[b])
            pltpu.async_copy(f_r.at[idxq_s.at[q]], fbh.at[b], gsem.at[b])
            pltpu.async_copy(ex.at[pl.ds(tbase + kk * C2, C2)], exb.at[b],
                             gsem.at[b])

        def gathers_wait(b):
            pltpu.make_async_copy(inv.at[pl.ds(0, C2)], invb.at[b],
                                  gsem.at[b]).wait()
            pltpu.make_async_copy(f_r.at[pl.ds(0, C2)], fbh.at[b],
                                  gsem.at[b]).wait()
            pltpu.make_async_copy(ex.at[pl.ds(0, C2)], exb.at[b],
                                  gsem.at[b]).wait()

        def scatter(pb, q):
            pltpu.async_copy(prod.at[pb], osh.at[idxq_d.at[q]], ssem.at[pb],
                             add=True)

        def scatter_wait(pb, q):
            pltpu.make_async_copy(prod.at[pb], osh.at[idxq_d.at[q]],
                                  ssem.at[pb]).wait()

        def compute(b, pb):
            @plsc.parallel_loop(0, C2, unroll=4)
            def row(i):
                al = exb[b, i, :] * invb[b, i, :]
                for hh in range(4):
                    bc = _bcast(al, col[hh])
                    v32 = fbh[b, i, pl.ds(hh * 32, 32)]
                    ev, od = plsc.unpack(v32, format=plsc.PackFormat.INTERLEAVED)
                    prod[pb, i, pl.ds(hh * 32, L)] = ev * bc
                    prod[pb, i, pl.ds(hh * 32 + L, L)] = od * bc

        for q in range(4):
            idx_load(q, q)
        idx_wait(0)
        gathers(0, 0, 0)
        idx_wait(1)
        gathers(1, 1, 1)

        def phase(k, j, p_is_dyn, p=None):
            b, q, pb = j % NB, j % NQ, j % 2
            gathers_wait(b)
            compute(b, pb)
            scatter(pb, q)
            jw = (j - 1) % GRP
            if j >= 1:
                scatter_wait(jw % 2, jw % NQ)
            elif p_is_dyn:
                @pl.when(p > 0)
                def _():
                    scatter_wait(jw % 2, jw % NQ)
            j2 = (j + 2) % GRP
            idx_wait(j2 % NQ)
            gathers(k + 2, j2 % NB, j2 % NQ)
            idx_load(k + 4, (j + 4) % NQ)

        def grp(p, _):
            k0 = p * GRP
            for j in range(GRP):
                phase(k0 + j, j, True, p)
            return 0
        lax.fori_loop(0, MAIN, grp, 0)

        for k in range(MAIN * GRP, NCH):
            j = k % GRP
            b, q, pb = j % NB, j % NQ, j % 2
            gathers_wait(b)
            compute(b, pb)
            scatter(pb, q)
            jw = (j - 1) % GRP
            scatter_wait(jw % 2, jw % NQ)
            if k + 2 < NCH:
                j2 = (j + 2) % GRP
                idx_wait(j2 % NQ)
                gathers(k + 2, j2 % NB, j2 % NQ)
            if k + 4 < NCH:
                idx_load(k + 4, (j + 4) % NQ)
        jl = (NCH - 1) % GRP
        scatter_wait(jl % 2, jl % NQ)

        plsc.subcore_barrier()
        pltpu.sync_copy(osh.at[pl.ds(s * NPT, NPT)], o_r.at[pl.ds(s * NPT, NPT)])


    for (srcr, dstr, exr, invr, fh0, fh1, oh0, oh1) in (
            (src_p, dst_p, ex_p, inv_p, fp0, fp1, op0, op1),
            (src_n, dst_n, ex_n, inv_n, fn0, fn1, on0, on1)):
        pltpu.sync_copy(z128, osh.at[pl.ds(s * NPT, NPT)])
        plsc.subcore_barrier()

        @pl.when(c == 0)
        def _():
            run(srcr, dstr, exr, invr, fh0, oh0, 0)

        @pl.when(c == 1)
        def _():
            run(srcr, dstr, exr, invr, fh1, oh1, 4)

        plsc.subcore_barrier()


def _pass2(src_p, dst_p, src_n, dst_n, ex_p, ex_n, inv_p, inv_n,
           fp0, fp1, fn0, fn1, z128):
    f = pl.kernel(
        _pass2_body,
        out_type=[jax.ShapeDtypeStruct((NP, HALF), jnp.float32)] * 4,
        mesh=_sc_mesh(),
        compiler_params=pltpu.CompilerParams(use_tc_tiling_on_sc=False,
                                             needs_layout_passes=False),
        scratch_types=[
            pltpu.VMEM((NQ, C2), jnp.int32),
            pltpu.VMEM((NQ, C2), jnp.int32),
            pltpu.VMEM((NB, C2, L), jnp.float32),
            pltpu.VMEM((NB, C2, L), jnp.float32),
            pltpu.VMEM((NB, C2, HALF), jnp.bfloat16),
            pltpu.VMEM((2, C2, HALF), jnp.float32),
            pltpu.VMEM_SHARED((NP, HALF), jnp.float32),
            pltpu.SemaphoreType.DMA((NQ,)),
            pltpu.SemaphoreType.DMA((NB,)),
            pltpu.SemaphoreType.DMA((2,)),
        ],
    )
    return f(src_p, dst_p, src_n, dst_n, ex_p, ex_n, inv_p, inv_n,
             fp0, fp1, fn0, fn1, z128)


def _mlp_body(op0_ref, op1_ref, on0_ref, on1_ref, m0_ref, m1_ref,
              bp_ref, bn_ref, w1_ref, b1_ref, w2_ref, b2_ref,
              hp_ref, hn_ref, hf_ref):
    m0 = m0_ref[...]
    m1 = m1_ref[...]
    hp = (jnp.dot(op0_ref[...], m0, preferred_element_type=jnp.float32)
          + jnp.dot(op1_ref[...], m1, preferred_element_type=jnp.float32)
          + bp_ref[0:1, :])
    hn = (jnp.dot(on0_ref[...], m0, preferred_element_type=jnp.float32)
          + jnp.dot(on1_ref[...], m1, preferred_element_type=jnp.float32)
          + bn_ref[0:1, :])
    hp_ref[...] = hp
    hn_ref[...] = hn
    w1 = w1_ref[...]
    z = (jnp.dot(hp, w1[0:HID, :], preferred_element_type=jnp.float32)
         + jnp.dot(hn, w1[HID:, :], preferred_element_type=jnp.float32)
         + b1_ref[0:1, :])
    z = jnp.maximum(z, 0.0)
    hf_ref[...] = jnp.dot(z, w2_ref[...], preferred_element_type=jnp.float32) + b2_ref[0:1, :]


def _mlp(op0, op1, on0, on1, m0, m1, bp, bn, w1, b1, w2, b2):
    R = 2000
    grid = (N // R,)
    bs_h = pl.BlockSpec((R, HALF), lambda i: (i, 0))
    bs_m = pl.BlockSpec((HALF, HID), lambda i: (0, 0))
    return pl.pallas_call(
        _mlp_body,
        grid=grid,
        in_specs=[bs_h, bs_h, bs_h, bs_h, bs_m, bs_m,
                  pl.BlockSpec((1, HID), lambda i: (0, 0)),
                  pl.BlockSpec((1, HID), lambda i: (0, 0)),
                  pl.BlockSpec((2 * HID, HID), lambda i: (0, 0)),
                  pl.BlockSpec((1, HID), lambda i: (0, 0)),
                  pl.BlockSpec((HID, OUT_DIM), lambda i: (0, 0)),
                  pl.BlockSpec((1, OUT_DIM), lambda i: (0, 0))],
        out_specs=[pl.BlockSpec((R, HID), lambda i: (i, 0)),
                   pl.BlockSpec((R, HID), lambda i: (i, 0)),
                   pl.BlockSpec((R, OUT_DIM), lambda i: (i, 0))],
        out_shape=[jax.ShapeDtypeStruct((N, HID), jnp.float32),
                   jax.ShapeDtypeStruct((N, HID), jnp.float32),
                   jax.ShapeDtypeStruct((N, OUT_DIM), jnp.float32)],
    )(op0, op1, on0, on1, m0, m1, bp, bn, w1, b1, w2, b2)


def kernel(features, pos_edge_index, neg_edge_index, W_pos, attn_l_pos,
           attn_r_pos, b_pos, W_neg, attn_l_neg, attn_r_neg, b_neg,
           W1, b1, W2, b2):
    src_p = pos_edge_index[0].astype(jnp.int32)
    dst_p = pos_edge_index[1].astype(jnp.int32)
    src_n = neg_edge_index[0].astype(jnp.int32)
    dst_n = neg_edge_index[1].astype(jnp.int32)

    eye = jnp.eye(H, L, dtype=jnp.float32)
    alp = (attn_l_pos[:, :, None] * eye[:, None, :]).reshape(HID, L)
    arp = (attn_r_pos[:, :, None] * eye[:, None, :]).reshape(HID, L)
    aln = (attn_l_neg[:, :, None] * eye[:, None, :]).reshape(HID, L)
    arn = (attn_r_neg[:, :, None] * eye[:, None, :]).reshape(HID, L)

    z16 = jnp.zeros((NPT, L), jnp.float32)
    z128 = jnp.zeros((NPT, HALF), jnp.float32)

    fp0, fp1, fn0, fn1, tlp, trp, tln, trn = _encode(
        features, W_pos, W_neg, alp, arp, aln, arn)

    ex_p, ex_n, inv_p, inv_n = _pass1(
        src_p, dst_p, src_n, dst_n, tlp, trp, tln, trn, z16)

    op0, op1, on0, on1 = _pass2(src_p, dst_p, src_n, dst_n, ex_p, ex_n,
                                inv_p, inv_n, fp0, fp1, fn0, fn1, z128)

    ow = [32 * (j // 32) + (2 * (j % 32) if (j % 32) < L else
                            2 * ((j % 32) - L) + 1) for j in range(HALF)]
    eye_h = jnp.eye(HID, dtype=jnp.float32)
    m0 = eye_h[jnp.array(ow)]
    m1 = eye_h[jnp.array([o + HALF for o in ow])]

    h_pos, h_neg, h_final = _mlp(
        op0, op1, on0, on1, m0, m1,
        b_pos.reshape(1, HID), b_neg.reshape(1, HID),
        W1, b1.reshape(1, HID), W2, b2.reshape(1, OUT_DIM))
    return (h_pos, h_neg, h_final)


# speedup vs baseline: 110.6692x; 1.0098x over previous
import functools

import jax
import jax.numpy as jnp
from jax import lax
from jax.experimental import pallas as pl
from jax.experimental.pallas import tpu as pltpu
from jax.experimental.pallas import tpu_sc as plsc

N = 10000
IN_DIM = 128
HID = 256
OUT_DIM = 128
H = 8
DH = HID // H
E = 160000

NC = 2
NS = 16
L = 16

NP = 10240
NPT = NP // NS
EPT = E // NS
C1 = 1000
C2 = 80
HALF = HID // 2


def _sc_mesh():
    return plsc.VectorSubcoreMesh(core_axis_name="c", subcore_axis_name="s",
                                  num_cores=NC, num_subcores=NS)


_BCAST_DNUMS = lax.GatherDimensionNumbers(
    offset_dims=(), collapsed_slice_dims=(0,), start_index_map=(0,))


def _bcast(v, idx):
    return lax.gather(v, idx[:, None], _BCAST_DNUMS, (1,),
                      mode=lax.GatherScatterMode.PROMISE_IN_BOUNDS)


def _encode_body(x_ref, wp_ref, wn_ref, alp_ref, arp_ref, aln_ref, arn_ref,
                 fp0_ref, fp1_ref, fn0_ref, fn1_ref,
                 tlp_ref, trp_ref, tln_ref, trn_ref):
    x = x_ref[...]
    fp = jnp.dot(x, wp_ref[...], preferred_element_type=jnp.float32)
    fn = jnp.dot(x, wn_ref[...], preferred_element_type=jnp.float32)
    fp0_ref[...] = fp[:, :HALF].astype(jnp.bfloat16)
    fp1_ref[...] = fp[:, HALF:].astype(jnp.bfloat16)
    fn0_ref[...] = fn[:, :HALF].astype(jnp.bfloat16)
    fn1_ref[...] = fn[:, HALF:].astype(jnp.bfloat16)
    tlp_ref[...] = jnp.dot(fp, alp_ref[...], preferred_element_type=jnp.float32)
    trp_ref[...] = jnp.dot(fp, arp_ref[...], preferred_element_type=jnp.float32)
    tln_ref[...] = jnp.dot(fn, aln_ref[...], preferred_element_type=jnp.float32)
    trn_ref[...] = jnp.dot(fn, arn_ref[...], preferred_element_type=jnp.float32)


def _encode(x, wp, wn, alp, arp, aln, arn):
    R = 2000
    grid = (N // R,)
    bs_x = pl.BlockSpec((R, IN_DIM), lambda i: (i, 0))
    bs_w = pl.BlockSpec((IN_DIM, HID), lambda i: (0, 0))
    bs_a = pl.BlockSpec((HID, L), lambda i: (0, 0))
    bs_f = pl.BlockSpec((R, HALF), lambda i: (i, 0))
    bs_t = pl.BlockSpec((R, L), lambda i: (i, 0))
    outs = [jax.ShapeDtypeStruct((N, HALF), jnp.bfloat16)] * 4 + \
           [jax.ShapeDtypeStruct((N, L), jnp.float32)] * 4
    return pl.pallas_call(
        _encode_body,
        grid=grid,
        in_specs=[bs_x, bs_w, bs_w, bs_a, bs_a, bs_a, bs_a],
        out_specs=[bs_f] * 4 + [bs_t] * 4,
        out_shape=outs,
    )(x, wp, wn, alp, arp, aln, arn)


C1N = EPT // C1
M1 = (C1N - 5) // 6


def _pass1_body(src_p, dst_p, src_n, dst_n, tl_p, tr_p, tl_n, tr_n, z16,
                ex_p, ex_n, inv_p, inv_n,
                idxq_s, idxq_d, rl, rr, fin_b, s_sh, isem, gsem, wsem, ssem):
    c = lax.axis_index("c")
    s = lax.axis_index("s")

    pltpu.sync_copy(z16, s_sh.at[pl.ds(s * NPT, NPT)])
    plsc.subcore_barrier()

    def run(src_r, dst_r, tl_r, tr_r, ex_r):
        tbase = s * EPT

        def idx_load(kk, q):
            pltpu.async_copy(src_r.at[pl.ds(tbase + kk * C1, C1)],
                             idxq_s.at[q], isem.at[q])
            pltpu.async_copy(dst_r.at[pl.ds(tbase + kk * C1, C1)],
                             idxq_d.at[q], isem.at[q])

        def idx_wait(q):
            pltpu.make_async_copy(src_r.at[pl.ds(0, C1)], idxq_s.at[q],
                                  isem.at[q]).wait()
            pltpu.make_async_copy(dst_r.at[pl.ds(0, C1)], idxq_d.at[q],
                                  isem.at[q]).wait()

        def gathers(b, q):
            pltpu.async_copy(tl_r.at[idxq_s.at[q]], rl.at[b], gsem.at[b])
            pltpu.async_copy(tr_r.at[idxq_d.at[q]], rr.at[b], gsem.at[b])

        def gathers_wait(b):
            pltpu.make_async_copy(tl_r.at[pl.ds(0, C1)], rl.at[b],
                                  gsem.at[b]).wait()
            pltpu.make_async_copy(tr_r.at[pl.ds(0, C1)], rr.at[b],
                                  gsem.at[b]).wait()

        def stores(kk, b, q):
            pltpu.async_copy(rl.at[b], ex_r.at[pl.ds(tbase + kk * C1, C1)],
                             wsem.at[b])
            pltpu.async_copy(rl.at[b], s_sh.at[idxq_d.at[q]], ssem.at[b],
                             add=True)

        def stores_wait(kk, b, q):
            pltpu.make_async_copy(rl.at[b], ex_r.at[pl.ds(tbase + kk * C1, C1)],
                                  wsem.at[b]).wait()
            pltpu.make_async_copy(rl.at[b], s_sh.at[idxq_d.at[q]],
                                  ssem.at[b]).wait()

        def compute(b):
            @plsc.parallel_loop(0, C1, unroll=8)
            def row(i):
                v = rl[b, i, :] + rr[b, i, :]
                rl[b, i, :] = jnp.exp(jnp.maximum(v, 0.2 * v))

        for q in range(4):
            idx_load(q, q)
        idx_wait(0)
        gathers(0, 0)
        idx_wait(1)
        gathers(1, 1)

        def phase(k, j, p_is_dyn, p=None):
            b, q = j % 3, j % 6
            gathers_wait(b)
            compute(b)
            stores(k, b, q)
            jw = (j - 1) % 6
            if j >= 1:
                stores_wait(k - 1, jw % 3, jw % 6)
            elif p_is_dyn:
                @pl.when(p > 0)
                def _():
                    stores_wait(k - 1, jw % 3, jw % 6)
            j2 = (j + 2) % 6
            idx_wait(j2 % 6)
            gathers(j2 % 3, j2 % 6)
            idx_load(k + 4, (j + 4) % 6)

        def grp(p, _):
            k0 = p * 6
            for j in range(6):
                phase(k0 + j, j, True, p)
            return 0
        lax.fori_loop(0, M1, grp, 0)

        for k in range(M1 * 6, C1N):
            j = k % 6
            b, q = j % 3, j % 6
            gathers_wait(b)
            compute(b)
            stores(k, b, q)
            jw = (j - 1) % 6
            if k >= 1:
                stores_wait(k - 1, jw % 3, jw % 6)
            if k + 2 < C1N:
                j2 = (j + 2) % 6
                idx_wait(j2 % 6)
                gathers(j2 % 3, j2 % 6)
            if k + 4 < C1N:
                idx_load(k + 4, (j + 4) % 6)
        jl = (C1N - 1) % 6
        stores_wait(C1N - 1, jl % 3, jl % 6)

    @pl.when(c == 0)
    def _():
        run(src_p, dst_p, tl_p, tr_p, ex_p)

    @pl.when(c == 1)
    def _():
        run(src_n, dst_n, tl_n, tr_n, ex_n)

    plsc.subcore_barrier()

    pltpu.sync_copy(s_sh.at[pl.ds(s * NPT, NPT)], fin_b)

    @plsc.parallel_loop(0, NPT, unroll=8)
    def fin(i):
        fin_b[i, :] = 1.0 / jnp.maximum(fin_b[i, :], 1e-16)

    @pl.when(c == 0)
    def _():
        pltpu.sync_copy(fin_b, inv_p.at[pl.ds(s * NPT, NPT)])

    @pl.when(c == 1)
    def _():
        pltpu.sync_copy(fin_b, inv_n.at[pl.ds(s * NPT, NPT)])


def _pass1(src_p, dst_p, src_n, dst_n, tl_p, tr_p, tl_n, tr_n, z16):
    f = pl.kernel(
        _pass1_body,
        out_type=[jax.ShapeDtypeStruct((E, L), jnp.float32),
                  jax.ShapeDtypeStruct((E, L), jnp.float32),
                  jax.ShapeDtypeStruct((NP, L), jnp.float32),
                  jax.ShapeDtypeStruct((NP, L), jnp.float32)],
        mesh=_sc_mesh(),
        compiler_params=pltpu.CompilerParams(use_tc_tiling_on_sc=False,
                                             needs_layout_passes=False),
        scratch_types=[
            pltpu.VMEM((6, C1), jnp.int32),
            pltpu.VMEM((6, C1), jnp.int32),
            pltpu.VMEM((3, C1, L), jnp.float32),
            pltpu.VMEM((3, C1, L), jnp.float32),
            pltpu.VMEM((NPT, L), jnp.float32),
            pltpu.VMEM_SHARED((NP, L), jnp.float32),
            pltpu.SemaphoreType.DMA((6,)),
            pltpu.SemaphoreType.DMA((3,)),
            pltpu.SemaphoreType.DMA((3,)),
            pltpu.SemaphoreType.DMA((3,)),
        ],
    )
    return f(src_p, dst_p, src_n, dst_n, tl_p, tr_p, tl_n, tr_n, z16)


NB = 3
NQ = 6
NCH = EPT // C2
GRP = 6
MAIN = (NCH - 5) // GRP


def _pass2_body(src_p, dst_p, src_n, dst_n, ex_p, ex_n, inv_p, inv_n,
                fp0, fp1, fn0, fn1, z128,
                op0, op1, on0, on1,
                idxq_s, idxq_d, exb, invb, fbh, prod, osh, isem, gsem, ssem):
    c = lax.axis_index("c")
    s = lax.axis_index("s")

    def run(src, dst, ex, inv, f_r, o_r, head_base):
        col = [jnp.full((L,), head_base + hh, jnp.int32) for hh in range(4)]
        tbase = s * EPT

        def idx_load(kk, q):
            pltpu.async_copy(src.at[pl.ds(tbase + kk * C2, C2)],
                             idxq_s.at[q], isem.at[q])
            pltpu.async_copy(dst.at[pl.ds(tbase + kk * C2, C2)],
                             idxq_d.at[q], isem.at[q])

        def idx_wait(q):
            pltpu.make_async_copy(src.at[pl.ds(0, C2)], idxq_s.at[q],
                                  isem.at[q]).wait()
            pltpu.make_async_copy(dst.at[pl.ds(0, C2)], idxq_d.at[q],
                                  isem.at[q]).wait()

        def gathers(kk, b, q):
            pltpu.async_copy(inv.at[idxq_d.at[q]], invb.at[b], gsem.at[b])
            pltpu.async_copy(f_r.at[idxq_s.at[q]], fbh.at[b], gsem.at[b])
            pltpu.async_copy(ex.at[pl.ds(tbase + kk * C2, C2)], exb.at[b],
                             gsem.at[b])

        def gathers_wait(b):
            pltpu.make_async_copy(inv.at[pl.ds(0, C2)], invb.at[b],
                                  gsem.at[b]).wait()
            pltpu.make_async_copy(f_r.at[pl.ds(0, C2)], fbh.at[b],
                                  gsem.at[b]).wait()
            pltpu.make_async_copy(ex.at[pl.ds(0, C2)], exb.at[b],
                                  gsem.at[b]).wait()

        def scatter(pb, q):
            pltpu.async_copy(prod.at[pb], osh.at[idxq_d.at[q]], ssem.at[pb],
                             add=True)

        def scatter_wait(pb, q):
            pltpu.make_async_copy(prod.at[pb], osh.at[idxq_d.at[q]],
                                  ssem.at[pb]).wait()

        def compute(b, pb):
            @plsc.parallel_loop(0, C2, unroll=4)
            def row(i):
                al = exb[b, i, :] * invb[b, i, :]
                for hh in range(4):
                    bc = _bcast(al, col[hh])
                    v32 = fbh[b, i, pl.ds(hh * 32, 32)]
                    ev, od = plsc.unpack(v32, format=plsc.PackFormat.INTERLEAVED)
                    prod[pb, i, pl.ds(hh * 32, L)] = ev * bc
                    prod[pb, i, pl.ds(hh * 32 + L, L)] = od * bc

        for q in range(4):
            idx_load(q, q)
        idx_wait(0)
        gathers(0, 0, 0)
        idx_wait(1)
        gathers(1, 1, 1)

        def phase(k, j, p_is_dyn, p=None):
            b, q, pb = j % NB, j % NQ, j % 2
            gathers_wait(b)
            compute(b, pb)
            scatter(pb, q)
            jw = (j - 1) % GRP
            if j >= 1:
                scatter_wait(jw % 2, jw % NQ)
            elif p_is_dyn:
                @pl.when(p > 0)
                def _():
                    scatter_wait(jw % 2, jw % NQ)
            j2 = (j + 2) % GRP
            idx_wait(j2 % NQ)
            gathers(k + 2, j2 % NB, j2 % NQ)
            idx_load(k + 4, (j + 4) % NQ)

        def grp(p, _):
            k0 = p * GRP
            for j in range(GRP):
                phase(k0 + j, j, True, p)
            return 0
        lax.fori_loop(0, MAIN, grp, 0)

        for k in range(MAIN * GRP, NCH):
            j = k % GRP
            b, q, pb = j % NB, j % NQ, j % 2
            gathers_wait(b)
            compute(b, pb)
            scatter(pb, q)
            jw = (j - 1) % GRP
            scatter_wait(jw % 2, jw % NQ)
            if k + 2 < NCH:
                j2 = (j + 2) % GRP
                idx_wait(j2 % NQ)
                gathers(k + 2, j2 % NB, j2 % NQ)
            if k + 4 < NCH:
                idx_load(k + 4, (j + 4) % NQ)
        jl = (NCH - 1) % GRP
        scatter_wait(jl % 2, jl % NQ)

        plsc.subcore_barrier()
        pltpu.sync_copy(osh.at[pl.ds(s * NPT, NPT)], o_r.at[pl.ds(s * NPT, NPT)])


    for (srcr, dstr, exr, invr, fh0, fh1, oh0, oh1) in (
            (src_p, dst_p, ex_p, inv_p, fp0, fp1, op0, op1),
            (src_n, dst_n, ex_n, inv_n, fn0, fn1, on0, on1)):
        pltpu.sync_copy(z128, osh.at[pl.ds(s * NPT, NPT)])
        plsc.subcore_barrier()

        @pl.when(c == 0)
        def _():
            run(srcr, dstr, exr, invr, fh0, oh0, 0)

        @pl.when(c == 1)
        def _():
            run(srcr, dstr, exr, invr, fh1, oh1, 4)

        plsc.subcore_barrier()


def _pass2(src_p, dst_p, src_n, dst_n, ex_p, ex_n, inv_p, inv_n,
           fp0, fp1, fn0, fn1, z128):
    f = pl.kernel(
        _pass2_body,
        out_type=[jax.ShapeDtypeStruct((NP, HALF), jnp.float32)] * 4,
        mesh=_sc_mesh(),
        compiler_params=pltpu.CompilerParams(use_tc_tiling_on_sc=False,
                                             needs_layout_passes=False),
        scratch_types=[
            pltpu.VMEM((NQ, C2), jnp.int32),
            pltpu.VMEM((NQ, C2), jnp.int32),
            pltpu.VMEM((NB, C2, L), jnp.float32),
            pltpu.VMEM((NB, C2, L), jnp.float32),
            pltpu.VMEM((NB, C2, HALF), jnp.bfloat16),
            pltpu.VMEM((2, C2, HALF), jnp.float32),
            pltpu.VMEM_SHARED((NP, HALF), jnp.float32),
            pltpu.SemaphoreType.DMA((NQ,)),
            pltpu.SemaphoreType.DMA((NB,)),
            pltpu.SemaphoreType.DMA((2,)),
        ],
    )
    return f(src_p, dst_p, src_n, dst_n, ex_p, ex_n, inv_p, inv_n,
             fp0, fp1, fn0, fn1, z128)


def _mlp_body(op0_ref, op1_ref, on0_ref, on1_ref, m0_ref, m1_ref,
              bp_ref, bn_ref, w1_ref, b1_ref, w2_ref, b2_ref,
              hp_ref, hn_ref, hf_ref):
    m0 = m0_ref[...]
    m1 = m1_ref[...]
    hp = (jnp.dot(op0_ref[...], m0, preferred_element_type=jnp.float32)
          + jnp.dot(op1_ref[...], m1, preferred_element_type=jnp.float32)
          + bp_ref[0:1, :])
    hn = (jnp.dot(on0_ref[...], m0, preferred_element_type=jnp.float32)
          + jnp.dot(on1_ref[...], m1, preferred_element_type=jnp.float32)
          + bn_ref[0:1, :])
    hp_ref[...] = hp
    hn_ref[...] = hn
    w1 = w1_ref[...]
    z = (jnp.dot(hp, w1[0:HID, :], preferred_element_type=jnp.float32)
         + jnp.dot(hn, w1[HID:, :], preferred_element_type=jnp.float32)
         + b1_ref[0:1, :])
    z = jnp.maximum(z, 0.0)
    hf_ref[...] = jnp.dot(z, w2_ref[...], preferred_element_type=jnp.float32) + b2_ref[0:1, :]


def _mlp(op0, op1, on0, on1, m0, m1, bp, bn, w1, b1, w2, b2):
    R = 2000
    grid = (N // R,)
    bs_h = pl.BlockSpec((R, HALF), lambda i: (i, 0))
    bs_m = pl.BlockSpec((HALF, HID), lambda i: (0, 0))
    return pl.pallas_call(
        _mlp_body,
        grid=grid,
        in_specs=[bs_h, bs_h, bs_h, bs_h, bs_m, bs_m,
                  pl.BlockSpec((1, HID), lambda i: (0, 0)),
                  pl.BlockSpec((1, HID), lambda i: (0, 0)),
                  pl.BlockSpec((2 * HID, HID), lambda i: (0, 0)),
                  pl.BlockSpec((1, HID), lambda i: (0, 0)),
                  pl.BlockSpec((HID, OUT_DIM), lambda i: (0, 0)),
                  pl.BlockSpec((1, OUT_DIM), lambda i: (0, 0))],
        out_specs=[pl.BlockSpec((R, HID), lambda i: (i, 0)),
                   pl.BlockSpec((R, HID), lambda i: (i, 0)),
                   pl.BlockSpec((R, OUT_DIM), lambda i: (i, 0))],
        out_shape=[jax.ShapeDtypeStruct((N, HID), jnp.float32),
                   jax.ShapeDtypeStruct((N, HID), jnp.float32),
                   jax.ShapeDtypeStruct((N, OUT_DIM), jnp.float32)],
    )(op0, op1, on0, on1, m0, m1, bp, bn, w1, b1, w2, b2)


def kernel(features, pos_edge_index, neg_edge_index, W_pos, attn_l_pos,
           attn_r_pos, b_pos, W_neg, attn_l_neg, attn_r_neg, b_neg,
           W1, b1, W2, b2):
    src_p = pos_edge_index[0].astype(jnp.int32)
    dst_p = pos_edge_index[1].astype(jnp.int32)
    src_n = neg_edge_index[0].astype(jnp.int32)
    dst_n = neg_edge_index[1].astype(jnp.int32)

    eye = jnp.eye(H, L, dtype=jnp.float32)
    alp = (attn_l_pos[:, :, None] * eye[:, None, :]).reshape(HID, L)
    arp = (attn_r_pos[:, :, None] * eye[:, None, :]).reshape(HID, L)
    aln = (attn_l_neg[:, :, None] * eye[:, None, :]).reshape(HID, L)
    arn = (attn_r_neg[:, :, None] * eye[:, None, :]).reshape(HID, L)

    z16 = jnp.zeros((NPT, L), jnp.float32)
    z128 = jnp.zeros((NPT, HALF), jnp.float32)

    fp0, fp1, fn0, fn1, tlp, trp, tln, trn = _encode(
        features, W_pos, W_neg, alp, arp, aln, arn)

    ex_p, ex_n, inv_p, inv_n = _pass1(
        src_p, dst_p, src_n, dst_n, tlp, trp, tln, trn, z16)

    op0, op1, on0, on1 = _pass2(src_p, dst_p, src_n, dst_n, ex_p, ex_n,
                                inv_p, inv_n, fp0, fp1, fn0, fn1, z128)

    ow = [32 * (j // 32) + (2 * (j % 32) if (j % 32) < L else
                            2 * ((j % 32) - L) + 1) for j in range(HALF)]
    eye_h = jnp.eye(HID, dtype=jnp.float32)
    m0 = eye_h[jnp.array(ow)]
    m1 = eye_h[jnp.array([o + HALF for o in ow])]

    h_pos, h_neg, h_final = _mlp(
        op0, op1, on0, on1, m0, m1,
        b_pos.reshape(1, HID), b_neg.reshape(1, HID),
        W1, b1.reshape(1, HID), W2, b2.reshape(1, OUT_DIM))
    return (h_pos, h_neg, h_final)
